# Initial kernel scaffold; baseline (speedup 1.0000x reference)
import jax
import jax.numpy as jnp
from jax.experimental import pallas as pl


def kernel(z, template_x, edge_index, mlp_w1, mlp_b1, mlp_w2, mlp_b2, mlp_w3, mlp_b3, gat1_w, gat1_as, gat1_ad, gat1_b, ln1_g, ln1_b, gat2_w, gat2_as, gat2_ad, gat2_b, ln2_g, ln2_b, gat3_w, gat3_as, gat3_ad, gat3_b):
    raise NotImplementedError("write your pallas kernel here")


# speedup vs baseline: 1.0818x; 1.0818x over previous
import jax
import jax.numpy as jnp
from jax.experimental import pallas as pl
from jax.experimental.pallas import tpu as pltpu

B = 2
N = 10000
E = 160000
HEADS = 4
HID = 64


def _mlp_body(z_ref, w1_ref, b1_ref, w2_ref, b2_ref, w3_ref, b3_ref, out_ref):
    z = z_ref[...]
    h = jnp.dot(z, w1_ref[...].T, preferred_element_type=jnp.float32) + b1_ref[...]
    h = jnp.where(h > 0, h, 0.2 * h)
    h = jnp.dot(h, w2_ref[...].T, preferred_element_type=jnp.float32) + b2_ref[...]
    h = jnp.where(h > 0, h, 0.2 * h)
    h = jnp.dot(h, w3_ref[...].T, preferred_element_type=jnp.float32) + b3_ref[...]
    out_ref[...] = h


def _style_mlp(z, w1, b1, w2, b2, w3, b3):
    return pl.pallas_call(
        _mlp_body,
        out_shape=jax.ShapeDtypeStruct((z.shape[0], w3.shape[0]), jnp.float32),
    )(z, w1, b1.reshape(1, -1), w2, b2.reshape(1, -1), w3, b3.reshape(1, -1))


def _layer_norm(x, g, b):
    mu = jnp.mean(x, axis=-1, keepdims=True)
    var = jnp.mean((x - mu) ** 2, axis=-1, keepdims=True)
    return g * (x - mu) / jnp.sqrt(var + 1e-5) + b


def _gat(h, src, dst, W, a_s, a_d, bias, heads, od, concat, n):
    x = (h @ W.T).reshape(-1, heads, od)
    al_s = jnp.sum(x * a_s[None, :, :], axis=-1)
    al_d = jnp.sum(x * a_d[None, :, :], axis=-1)
    alpha = jax.nn.leaky_relu(al_s[src] + al_d[dst], 0.2)
    ex = jnp.exp(alpha)
    den = jax.ops.segment_sum(ex, dst, num_segments=n)
    coef = ex / (den[dst] + 1e-16)
    out = jax.ops.segment_sum(x[src] * coef[:, :, None], dst, num_segments=n)
    if concat:
        out = out.reshape(n, heads * od)
    else:
        out = jnp.mean(out, axis=1)
    return out + bias


def kernel(z, template_x, edge_index, mlp_w1, mlp_b1, mlp_w2, mlp_b2, mlp_w3, mlp_b3, gat1_w, gat1_as, gat1_ad, gat1_b, ln1_g, ln1_b, gat2_w, gat2_as, gat2_ad, gat2_b, ln2_g, ln2_b, gat3_w, gat3_as, gat3_ad, gat3_b):
    offs = jnp.arange(B, dtype=edge_index.dtype) * N
    src = (edge_index[0][None, :] + offs[:, None]).reshape(-1)
    dst = (edge_index[1][None, :] + offs[:, None]).reshape(-1)
    Xb = jnp.tile(template_x, (B, 1))
    s = _style_mlp(z, mlp_w1, mlp_b1, mlp_w2, mlp_b2, mlp_w3, mlp_b3)
    s_exp = jnp.repeat(s, N, axis=0)
    H0 = jnp.concatenate([Xb, s_exp], axis=1)
    Nb = B * N
    H1 = jax.nn.leaky_relu(_gat(H0, src, dst, gat1_w, gat1_as, gat1_ad, gat1_b, HEADS, HID, True, Nb), 0.2)
    H1n = _layer_norm(H1, ln1_g, ln1_b)
    H2 = jax.nn.leaky_relu(_gat(H1n, src, dst, gat2_w, gat2_as, gat2_ad, gat2_b, HEADS, HID, True, Nb), 0.2)
    H2 = H2 + H1n
    H2n = _layer_norm(H2, ln2_g, ln2_b)
    Hout = _gat(H2n, src, dst, gat3_w, gat3_as, gat3_ad, gat3_b, 1, 13, False, Nb)
    delta_p = Hout[:, :3]
    delta_f = Hout[:, 3:]
    X_gen = Xb + delta_f
    p_gen = Xb[:, :3] + delta_p
    X_gen = jnp.concatenate([p_gen, X_gen[:, 3:]], axis=1)
    EF = p_gen[dst] - p_gen[src]
    return X_gen, p_gen, EF


# speedup vs baseline: 4.6521x; 4.3001x over previous
import functools

import jax
import jax.numpy as jnp
from jax.experimental import pallas as pl
from jax.experimental.pallas import tpu as pltpu

B = 2
N = 10000
E = 160000
NB = B * N
EB = B * E
HEADS = 4
HID = 64
NF = 10
STYLE = 118
GIN = NF + STYLE
F1 = HEADS * HID
NBLK = 400
GRID = NB // NBLK


def _leaky(x):
    return jnp.where(x > 0, x, 0.2 * x)


def _mlp_body(z_ref, w1_ref, b1_ref, w2_ref, b2_ref, w3_ref, b3_ref, out_ref):
    h = jnp.dot(z_ref[...], w1_ref[...].T, preferred_element_type=jnp.float32) + b1_ref[...]
    h = _leaky(h)
    h = jnp.dot(h, w2_ref[...].T, preferred_element_type=jnp.float32) + b2_ref[...]
    h = _leaky(h)
    out_ref[...] = jnp.dot(h, w3_ref[...].T, preferred_element_type=jnp.float32) + b3_ref[...]


def _style_mlp(z, w1, b1, w2, b2, w3, b3):
    return pl.pallas_call(
        _mlp_body,
        out_shape=jax.ShapeDtypeStruct((z.shape[0], w3.shape[0]), jnp.float32),
    )(z, w1, b1.reshape(1, -1), w2, b2.reshape(1, -1), w3, b3.reshape(1, -1))


def _tc0_body(xb_ref, s_ref, w1h_ref, as_ref, ad_ref, xt_ref, als_ref, ald_ref):
    h0 = jnp.concatenate([xb_ref[...], s_ref[...]], axis=1)
    als_cols, ald_cols = [], []
    for h in range(HEADS):
        xh = jnp.dot(h0, w1h_ref[h].T, preferred_element_type=jnp.float32)
        xt_ref[h] = xh
        als_cols.append(jnp.sum(xh * as_ref[h][None, :], axis=1, keepdims=True))
        ald_cols.append(jnp.sum(xh * ad_ref[h][None, :], axis=1, keepdims=True))
    als_ref[...] = jnp.concatenate(als_cols, axis=1)
    ald_ref[...] = jnp.concatenate(ald_cols, axis=1)


def _tc0(xb, s_exp, w1_heads, a_s, a_d):
    return pl.pallas_call(
        _tc0_body,
        grid=(GRID,),
        in_specs=[
            pl.BlockSpec((NBLK, NF), lambda i: (i, 0)),
            pl.BlockSpec((NBLK, STYLE), lambda i: (i, 0)),
            pl.BlockSpec((HEADS, HID, GIN), lambda i: (0, 0, 0)),
            pl.BlockSpec((HEADS, HID), lambda i: (0, 0)),
            pl.BlockSpec((HEADS, HID), lambda i: (0, 0)),
        ],
        out_specs=[
            pl.BlockSpec((HEADS, NBLK, HID), lambda i: (0, i, 0)),
            pl.BlockSpec((NBLK, HEADS), lambda i: (i, 0)),
            pl.BlockSpec((NBLK, HEADS), lambda i: (i, 0)),
        ],
        out_shape=[
            jax.ShapeDtypeStruct((HEADS, NB, HID), jnp.float32),
            jax.ShapeDtypeStruct((NB, HEADS), jnp.float32),
            jax.ShapeDtypeStruct((NB, HEADS), jnp.float32),
        ],
    )(xb, s_exp, w1_heads, a_s, a_d)


def _tcmid_body(out_t_ref, den_ref, bias_ref, g_ref, b_ref, res_ref,
                wh_ref, as_ref, ad_ref, hn_ref, xt_ref, als_ref, ald_ref,
                *, residual):
    cols = []
    for h in range(HEADS):
        cols.append(out_t_ref[h] / (den_ref[:, h][:, None] + 1e-16))
    hcat = jnp.concatenate(cols, axis=1) + bias_ref[...]
    hcat = _leaky(hcat)
    if residual:
        hcat = hcat + res_ref[...]
    mu = jnp.mean(hcat, axis=-1, keepdims=True)
    var = jnp.mean((hcat - mu) ** 2, axis=-1, keepdims=True)
    hn = g_ref[...] * (hcat - mu) / jnp.sqrt(var + 1e-5) + b_ref[...]
    hn_ref[...] = hn
    nh = xt_ref.shape[0]
    als_cols, ald_cols = [], []
    for h in range(nh):
        xh = jnp.dot(hn, wh_ref[h].T, preferred_element_type=jnp.float32)
        xt_ref[h] = xh
        als_cols.append(jnp.sum(xh * as_ref[h][None, :], axis=1, keepdims=True))
        ald_cols.append(jnp.sum(xh * ad_ref[h][None, :], axis=1, keepdims=True))
    als_ref[...] = jnp.concatenate(als_cols, axis=1)
    ald_ref[...] = jnp.concatenate(ald_cols, axis=1)


def _tcmid(out_t, den, bias, g, b, res, w_heads, a_s, a_d, residual):
    nh, od = w_heads.shape[0], w_heads.shape[1]
    body = functools.partial(_tcmid_body, residual=residual)
    return pl.pallas_call(
        body,
        grid=(GRID,),
        in_specs=[
            pl.BlockSpec((HEADS, NBLK, HID), lambda i: (0, i, 0)),
            pl.BlockSpec((NBLK, HEADS), lambda i: (i, 0)),
            pl.BlockSpec((1, F1), lambda i: (0, 0)),
            pl.BlockSpec((1, F1), lambda i: (0, 0)),
            pl.BlockSpec((1, F1), lambda i: (0, 0)),
            pl.BlockSpec((NBLK, F1), lambda i: (i, 0)),
            pl.BlockSpec((nh, od, F1), lambda i: (0, 0, 0)),
            pl.BlockSpec((nh, od), lambda i: (0, 0)),
            pl.BlockSpec((nh, od), lambda i: (0, 0)),
        ],
        out_specs=[
            pl.BlockSpec((NBLK, F1), lambda i: (i, 0)),
            pl.BlockSpec((nh, NBLK, od), lambda i: (0, i, 0)),
            pl.BlockSpec((NBLK, nh), lambda i: (i, 0)),
            pl.BlockSpec((NBLK, nh), lambda i: (i, 0)),
        ],
        out_shape=[
            jax.ShapeDtypeStruct((NB, F1), jnp.float32),
            jax.ShapeDtypeStruct((nh, NB, od), jnp.float32),
            jax.ShapeDtypeStruct((NB, nh), jnp.float32),
            jax.ShapeDtypeStruct((NB, nh), jnp.float32),
        ],
    )(out_t, den, bias, g, b, res, w_heads, a_s, a_d)


def _tc3_body(outp_ref, denp_ref, b3_ref, xb_ref, xgen_ref, p3_ref, ppad_ref):
    hout = outp_ref[0] / (denp_ref[...] + 1e-16) + b3_ref[...]
    xb = xb_ref[...]
    p_gen = xb[:, :3] + hout[:, :3]
    xtail = xb[:, 3:] + hout[:, 6:13]
    xgen_ref[...] = jnp.concatenate([p_gen, xtail], axis=1)
    p3_ref[...] = p_gen
    ppad_ref[...] = jnp.concatenate(
        [p_gen, jnp.zeros((p_gen.shape[0], 1), jnp.float32)], axis=1)


def _tc3(out3_t, den3, b3pad, xb):
    return pl.pallas_call(
        _tc3_body,
        grid=(GRID,),
        in_specs=[
            pl.BlockSpec((1, NBLK, 16), lambda i: (0, i, 0)),
            pl.BlockSpec((NBLK, 1), lambda i: (i, 0)),
            pl.BlockSpec((1, 16), lambda i: (0, 0)),
            pl.BlockSpec((NBLK, NF), lambda i: (i, 0)),
        ],
        out_specs=[
            pl.BlockSpec((NBLK, NF), lambda i: (i, 0)),
            pl.BlockSpec((NBLK, 3), lambda i: (i, 0)),
            pl.BlockSpec((NBLK, 4), lambda i: (i, 0)),
        ],
        out_shape=[
            jax.ShapeDtypeStruct((NB, NF), jnp.float32),
            jax.ShapeDtypeStruct((NB, 3), jnp.float32),
            jax.ShapeDtypeStruct((NB, 4), jnp.float32),
        ],
    )(out3_t, den3, b3pad, xb)


def _edge_phase(x_t, als, ald, src, dst):
    nh = x_t.shape[0]
    ex = jnp.exp(_leaky(als[src] + ald[dst]))
    den = jax.ops.segment_sum(ex, dst, num_segments=NB)
    out_t = jnp.stack([
        jax.ops.segment_sum(ex[:, h][:, None] * x_t[h][src], dst, num_segments=NB)
        for h in range(nh)])
    return out_t, den


def kernel(z, template_x, edge_index, mlp_w1, mlp_b1, mlp_w2, mlp_b2, mlp_w3, mlp_b3, gat1_w, gat1_as, gat1_ad, gat1_b, ln1_g, ln1_b, gat2_w, gat2_as, gat2_ad, gat2_b, ln2_g, ln2_b, gat3_w, gat3_as, gat3_ad, gat3_b):
    f = jnp.float32
    offs = jnp.arange(B, dtype=edge_index.dtype) * N
    src = (edge_index[0][None, :] + offs[:, None]).reshape(-1)
    dst = (edge_index[1][None, :] + offs[:, None]).reshape(-1)
    xb = jnp.concatenate([template_x, template_x], axis=0)

    w1_heads = gat1_w.reshape(HEADS, HID, GIN)
    w2_heads = gat2_w.reshape(HEADS, HID, F1)
    w3pad = jnp.concatenate([gat3_w, jnp.zeros((3, F1), f)], axis=0).reshape(1, 16, F1)
    as3pad = jnp.concatenate([gat3_as, jnp.zeros((1, 3), f)], axis=1)
    ad3pad = jnp.concatenate([gat3_ad, jnp.zeros((1, 3), f)], axis=1)
    b3pad = jnp.concatenate([gat3_b, jnp.zeros((3,), f)]).reshape(1, 16)

    s = _style_mlp(z, mlp_w1, mlp_b1, mlp_w2, mlp_b2, mlp_w3, mlp_b3)
    s_exp = jnp.repeat(s, N, axis=0)

    x1_t, als1, ald1 = _tc0(xb, s_exp, w1_heads, gat1_as, gat1_ad)
    out1_t, den1 = _edge_phase(x1_t, als1, ald1, src, dst)

    h1n, x2_t, als2, ald2 = _tcmid(
        out1_t, den1, gat1_b.reshape(1, F1), ln1_g.reshape(1, F1),
        ln1_b.reshape(1, F1), jnp.zeros((NB, F1), f), w2_heads,
        gat2_as, gat2_ad, residual=False)
    out2_t, den2 = _edge_phase(x2_t, als2, ald2, src, dst)

    _, x3_t, als3, ald3 = _tcmid(
        out2_t, den2, gat2_b.reshape(1, F1), ln2_g.reshape(1, F1),
        ln2_b.reshape(1, F1), h1n, w3pad, as3pad.reshape(1, 16),
        ad3pad.reshape(1, 16), residual=True)
    out3_t, den3 = _edge_phase(x3_t, als3, ald3, src, dst)

    x_gen, p3, ppad = _tc3(out3_t, den3, b3pad, xb)
    ef = (ppad[dst] - ppad[src])[:, :3]
    return x_gen, p3, ef


# speedup vs baseline: 23.7504x; 5.1053x over previous
import functools

import jax
import jax.numpy as jnp
from jax import lax
from jax.experimental import pallas as pl
from jax.experimental.pallas import tpu as pltpu
from jax.experimental.pallas import tpu_sc as plsc

B = 2
N = 10000
E = 160000
NB = B * N
EB = B * E
HEADS = 4
HID = 64
NF = 10
STYLE = 118
GIN = NF + STYLE
F1 = HEADS * HID
NBLK = 400
GRID = NB // NBLK
NPG = N // NBLK

CHK = 80
ECHUNKS = EB // CHK
NCORES = 2
NSUB = 16
ZW = 2000
ZT = NB // ZW


def _leaky(x):
    return jnp.where(x > 0, x, 0.2 * x)


def _mlp_body(z_ref, w1_ref, b1_ref, w2_ref, b2_ref, w3_ref, b3_ref, out_ref):
    h = jnp.dot(z_ref[...], w1_ref[...].T, preferred_element_type=jnp.float32) + b1_ref[...]
    h = _leaky(h)
    h = jnp.dot(h, w2_ref[...].T, preferred_element_type=jnp.float32) + b2_ref[...]
    h = _leaky(h)
    out_ref[...] = jnp.dot(h, w3_ref[...].T, preferred_element_type=jnp.float32) + b3_ref[...]


def _style_mlp(z, w1, b1, w2, b2, w3, b3):
    return pl.pallas_call(
        _mlp_body,
        out_shape=jax.ShapeDtypeStruct((z.shape[0], w3.shape[0]), jnp.float32),
    )(z, w1, b1.reshape(1, -1), w2, b2.reshape(1, -1), w3, b3.reshape(1, -1))


def _tc0_body(xb_ref, s_ref, w1h_ref, as_ref, ad_ref, xt_ref, als_ref, ald_ref):
    g = pl.program_id(0) // NPG
    srow = s_ref[pl.ds(g, 1)]
    h0 = jnp.concatenate(
        [xb_ref[...], jnp.broadcast_to(srow, (NBLK, STYLE))], axis=1)
    als_cols, ald_cols = [], []
    for h in range(HEADS):
        xh = jnp.dot(h0, w1h_ref[h].T, preferred_element_type=jnp.float32)
        xt_ref[h] = xh
        als_cols.append(jnp.sum(xh * as_ref[h][None, :], axis=1, keepdims=True))
        ald_cols.append(jnp.sum(xh * ad_ref[h][None, :], axis=1, keepdims=True))
    als_ref[...] = jnp.concatenate(als_cols, axis=1)
    ald_ref[...] = jnp.concatenate(ald_cols, axis=1)


def _tc0(template_x, s, w1_heads, a_s, a_d):
    return pl.pallas_call(
        _tc0_body,
        grid=(GRID,),
        in_specs=[
            pl.BlockSpec((NBLK, NF), lambda i: (i % NPG, 0)),
            pl.BlockSpec((B, STYLE), lambda i: (0, 0)),
            pl.BlockSpec((HEADS, HID, GIN), lambda i: (0, 0, 0)),
            pl.BlockSpec((HEADS, HID), lambda i: (0, 0)),
            pl.BlockSpec((HEADS, HID), lambda i: (0, 0)),
        ],
        out_specs=[
            pl.BlockSpec((HEADS, NBLK, HID), lambda i: (0, i, 0)),
            pl.BlockSpec((NBLK, HEADS), lambda i: (i, 0)),
            pl.BlockSpec((NBLK, HEADS), lambda i: (i, 0)),
        ],
        out_shape=[
            jax.ShapeDtypeStruct((HEADS, NB, HID), jnp.float32),
            jax.ShapeDtypeStruct((NB, HEADS), jnp.float32),
            jax.ShapeDtypeStruct((NB, HEADS), jnp.float32),
        ],
    )(template_x, s, w1_heads, a_s, a_d)


def _tcmid_body(fused_ref, bias_ref, g_ref, b_ref, res_ref,
                wh_ref, as_ref, ad_ref, hn_ref, xt_ref, als_ref, ald_ref,
                *, residual):
    cols = []
    for h in range(HEADS):
        fh = fused_ref[h]
        cols.append(fh[:, :HID] / (fh[:, HID:HID + 1] + 1e-16))
    hcat = jnp.concatenate(cols, axis=1) + bias_ref[...]
    hcat = _leaky(hcat)
    if residual:
        hcat = hcat + res_ref[...]
    mu = jnp.mean(hcat, axis=-1, keepdims=True)
    var = jnp.mean((hcat - mu) ** 2, axis=-1, keepdims=True)
    hn = g_ref[...] * (hcat - mu) / jnp.sqrt(var + 1e-5) + b_ref[...]
    hn_ref[...] = hn
    nh = xt_ref.shape[0]
    als_cols, ald_cols = [], []
    for h in range(nh):
        xh = jnp.dot(hn, wh_ref[h].T, preferred_element_type=jnp.float32)
        xt_ref[h] = xh
        als_cols.append(jnp.sum(xh * as_ref[h][None, :], axis=1, keepdims=True))
        ald_cols.append(jnp.sum(xh * ad_ref[h][None, :], axis=1, keepdims=True))
    als_ref[...] = jnp.concatenate(als_cols, axis=1)
    ald_ref[...] = jnp.concatenate(ald_cols, axis=1)


def _tcmid(fused, bias, g, b, res, w_heads, a_s, a_d, residual):
    nh, od = w_heads.shape[0], w_heads.shape[1]
    body = functools.partial(_tcmid_body, residual=residual)
    in_specs = [
        pl.BlockSpec((HEADS, NBLK, HID + 16), lambda i: (0, i, 0)),
        pl.BlockSpec((1, F1), lambda i: (0, 0)),
        pl.BlockSpec((1, F1), lambda i: (0, 0)),
        pl.BlockSpec((1, F1), lambda i: (0, 0)),
        pl.BlockSpec((NBLK, F1), lambda i: (i, 0)),
        pl.BlockSpec((nh, od, F1), lambda i: (0, 0, 0)),
        pl.BlockSpec((nh, od), lambda i: (0, 0)),
        pl.BlockSpec((nh, od), lambda i: (0, 0)),
    ]
    return pl.pallas_call(
        body,
        grid=(GRID,),
        in_specs=in_specs,
        out_specs=[
            pl.BlockSpec((NBLK, F1), lambda i: (i, 0)),
            pl.BlockSpec((nh, NBLK, od), lambda i: (0, i, 0)),
            pl.BlockSpec((NBLK, nh), lambda i: (i, 0)),
            pl.BlockSpec((NBLK, nh), lambda i: (i, 0)),
        ],
        out_shape=[
            jax.ShapeDtypeStruct((NB, F1), jnp.float32),
            jax.ShapeDtypeStruct((nh, NB, od), jnp.float32),
            jax.ShapeDtypeStruct((NB, nh), jnp.float32),
            jax.ShapeDtypeStruct((NB, nh), jnp.float32),
        ],
    )(fused, bias, g, b, res, w_heads, a_s, a_d)


def _tc3_body(f_ref, b3_ref, xb_ref, xgen_ref, p3_ref, p16_ref):
    sblk = f_ref[0] + f_ref[1]
    hout = sblk[:, :16] / (sblk[:, 16:17] + 1e-16) + b3_ref[...]
    xb = xb_ref[...]
    p_gen = xb[:, :3] + hout[:, :3]
    xtail = xb[:, 3:] + hout[:, 6:13]
    xgen_ref[...] = jnp.concatenate([p_gen, xtail], axis=1)
    p3_ref[...] = p_gen
    p16_ref[...] = jnp.concatenate(
        [p_gen, jnp.zeros((p_gen.shape[0], 13), jnp.float32)], axis=1)


def _tc3(fused3, b3pad, template_x):
    return pl.pallas_call(
        _tc3_body,
        grid=(GRID,),
        in_specs=[
            pl.BlockSpec((2, NBLK, 32), lambda i: (0, i, 0)),
            pl.BlockSpec((1, 16), lambda i: (0, 0)),
            pl.BlockSpec((NBLK, NF), lambda i: (i % NPG, 0)),
        ],
        out_specs=[
            pl.BlockSpec((NBLK, NF), lambda i: (i, 0)),
            pl.BlockSpec((NBLK, 3), lambda i: (i, 0)),
            pl.BlockSpec((NBLK, 16), lambda i: (i, 0)),
        ],
        out_shape=[
            jax.ShapeDtypeStruct((NB, NF), jnp.float32),
            jax.ShapeDtypeStruct((NB, 3), jnp.float32),
            jax.ShapeDtypeStruct((NB, 16), jnp.float32),
        ],
    )(fused3, b3pad, template_x)


_SC_PARAMS = pltpu.CompilerParams(
    use_tc_tiling_on_sc=False, needs_layout_passes=False)
BQ = 25
BE = BQ * CHK
NBATCH = EB // BE


def _make_ex_sc(heads, edge_split):
    hpc = 1 if edge_split else heads // NCORES
    nchk = ECHUNKS // (NCORES * NSUB) if edge_split else ECHUNKS // NSUB
    slots = NCORES * NSUB if edge_split else heads * NSUB
    mesh = plsc.VectorSubcoreMesh(core_axis_name="c", subcore_axis_name="s")

    @functools.partial(
        pl.kernel,
        mesh=mesh,
        out_type=jax.ShapeDtypeStruct((slots, nchk, CHK), jnp.float32),
        compiler_params=_SC_PARAMS,
        scratch_types=[
            pltpu.VMEM((nchk, CHK), jnp.int32),
            pltpu.VMEM((nchk, CHK), jnp.int32),
            pltpu.VMEM((NB,), jnp.float32),
            pltpu.VMEM((NB,), jnp.float32),
            pltpu.VMEM((nchk, CHK), jnp.float32),
        ],
    )
    def ex_kernel(als_ref, ald_ref, srcm_ref, dstm_ref, out_ref,
                  src_v, dst_v, als_v, ald_v, exb_v):
        c = lax.axis_index("c")
        s = lax.axis_index("s")
        tile = c * NSUB + s if edge_split else s
        pltpu.sync_copy(srcm_ref.at[tile], src_v)
        pltpu.sync_copy(dstm_ref.at[tile], dst_v)
        for slot in range(hpc):
            if edge_split:
                hoff = jnp.int32(0)
                oslot = tile
            else:
                h = c + 2 * slot
                hoff = h * NB
                oslot = h * NSUB + s
            pltpu.sync_copy(als_ref.at[pl.ds(hoff, NB)], als_v)
            pltpu.sync_copy(ald_ref.at[pl.ds(hoff, NB)], ald_v)

            def cbody(j, carry):
                for k in range(CHK // 16):
                    s16 = src_v[j, pl.ds(16 * k, 16)]
                    d16 = dst_v[j, pl.ds(16 * k, 16)]
                    a = (plsc.load_gather(als_v, [s16])
                         + plsc.load_gather(ald_v, [d16]))
                    exb_v[j, pl.ds(16 * k, 16)] = jnp.exp(_leaky(a))
                return carry
            lax.fori_loop(0, nchk, cbody, 0)
            pltpu.sync_copy(exb_v, out_ref.at[oslot])

    return ex_kernel


def _make_scatter_sc(heads, d, edge_split):
    roww = d + 16
    hpc = 1 if edge_split else heads // NCORES
    nbat = NBATCH // (NCORES * NSUB) if edge_split else NBATCH // NSUB
    mesh = plsc.VectorSubcoreMesh(core_axis_name="c", subcore_axis_name="s")
    out_rows = 2 * NB if edge_split else heads * NB

    @functools.partial(
        pl.kernel,
        mesh=mesh,
        out_type=jax.ShapeDtypeStruct((out_rows, roww), jnp.float32),
        compiler_params=_SC_PARAMS,
        scratch_types=[
            pltpu.VMEM((BQ, CHK), jnp.int32),
            pltpu.VMEM((BQ, CHK), jnp.int32),
            pltpu.VMEM((BQ, CHK), jnp.int32),
            pltpu.VMEM((BE,), jnp.float32),
            pltpu.VMEM((CHK, d), jnp.float32),
            pltpu.VMEM((CHK, roww), jnp.float32),
            pltpu.VMEM_SHARED((NB, roww), jnp.float32),
            pltpu.SemaphoreType.DMA,
        ],
    )
    def scatter_kernel(x_ref, exm_ref, srcb_ref, dstb_ref, z_ref,
                       out_ref, src_v, dst_v, gidx_v, ex_v, rows_v,
                       srow_v, acc, sem):
        c = lax.axis_index("c")
        s = lax.axis_index("s")
        tile = c * NSUB + s if edge_split else s

        for slot in range(hpc):
            if edge_split:
                hoff = jnp.int32(0)
                out_base = c * NB
                exslot0 = tile * nbat
            else:
                h = c + 2 * slot
                hoff = h * NB
                out_base = h * NB
                exslot0 = (h * NSUB + s) * nbat

            @pl.when(s < ZT)
            def _zero():
                pltpu.sync_copy(z_ref.at[s], acc.at[pl.ds(s * ZW, ZW)])

            plsc.subcore_barrier()

            def bbody(b, carry):
                gb = tile * nbat + b
                pltpu.sync_copy(srcb_ref.at[gb], src_v)
                pltpu.sync_copy(dstb_ref.at[gb], dst_v)
                pltpu.sync_copy(exm_ref.at[exslot0 + b], ex_v)

                def gbody(j, gc):
                    for k in range(CHK // 16):
                        gidx_v[j, pl.ds(16 * k, 16)] = (
                            src_v[j, pl.ds(16 * k, 16)] + hoff)
                    return gc
                lax.fori_loop(0, BQ, gbody, 0)

                def cbody(j, cc):
                    cp = pltpu.async_copy(x_ref.at[gidx_v.at[j]], rows_v, sem)
                    cp.wait()

                    def rbody(e, rc):
                        erow = jnp.full((16,), 0, jnp.int32) + (j * CHK + e)
                        bc = plsc.load_gather(ex_v, [erow])
                        for f in range(d // 16):
                            srow_v[e, pl.ds(16 * f, 16)] = (
                                rows_v[e, pl.ds(16 * f, 16)] * bc)
                        srow_v[e, pl.ds(d, 16)] = bc
                        return rc
                    lax.fori_loop(0, CHK, rbody, 0)
                    pltpu.sync_copy(srow_v, acc.at[dst_v.at[j]], add=True)
                    return cc
                lax.fori_loop(0, BQ, cbody, 0)
                return carry
            lax.fori_loop(0, nbat, bbody, 0)
            plsc.subcore_barrier()

            @pl.when(s < ZT)
            def _writeout():
                pltpu.sync_copy(acc.at[pl.ds(s * ZW, ZW)],
                                out_ref.at[pl.ds(out_base + s * ZW, ZW)])

            plsc.subcore_barrier()

    return scatter_kernel


_ex4 = _make_ex_sc(HEADS, edge_split=False)
_ex1 = _make_ex_sc(1, edge_split=True)
_scat4 = _make_scatter_sc(HEADS, HID, edge_split=False)
_scat1 = _make_scatter_sc(1, 16, edge_split=True)


def _edge4(x_flat, als_f, ald_f, srcm16, dstm16, srcb, dstb, zeros):
    exm = _ex4(als_f, ald_f, srcm16, dstm16)
    return _scat4(x_flat, exm.reshape(-1, BE), srcb, dstb, zeros)


def _edge1(x_flat, als_f, ald_f, srcm32, dstm32, srcb, dstb, zeros):
    exm = _ex1(als_f, ald_f, srcm32, dstm32)
    return _scat1(x_flat, exm.reshape(-1, BE), srcb, dstb, zeros)


_EF_NCHK = ECHUNKS // (NCORES * NSUB)


@functools.partial(
    pl.kernel,
    mesh=plsc.VectorSubcoreMesh(core_axis_name="c", subcore_axis_name="s"),
    out_type=jax.ShapeDtypeStruct((EB, 16), jnp.float32),
    compiler_params=pltpu.CompilerParams(use_tc_tiling_on_sc=False, needs_layout_passes=False),
    scratch_types=[
        pltpu.VMEM((_EF_NCHK, CHK), jnp.int32),
        pltpu.VMEM((_EF_NCHK, CHK), jnp.int32),
        pltpu.VMEM((CHK, 16), jnp.float32),
        pltpu.VMEM((CHK, 16), jnp.float32),
        pltpu.VMEM((CHK, 16), jnp.float32),
        pltpu.SemaphoreType.DMA,
        pltpu.SemaphoreType.DMA,
    ],
)
def _ef_kernel(p_ref, srcm_ref, dstm_ref, out_ref, src_v, dst_v,
               rs_v, rd_v, diff_v, sem_s, sem_d):
    c = lax.axis_index("c")
    s = lax.axis_index("s")
    iota = lax.iota(jnp.int32, 16)
    w = c * NSUB + s
    cb = w * _EF_NCHK
    pltpu.sync_copy(srcm_ref.at[w], src_v)
    pltpu.sync_copy(dstm_ref.at[w], dst_v)

    def cbody(j, carry):
        cps = pltpu.async_copy(p_ref.at[src_v.at[j]], rs_v, sem_s)
        cpd = pltpu.async_copy(p_ref.at[dst_v.at[j]], rd_v, sem_d)
        cps.wait()
        cpd.wait()

        def rbody(e, rc):
            diff_v[e] = rd_v[e] - rs_v[e]
            return rc
        lax.fori_loop(0, CHK, rbody, 0)
        pltpu.sync_copy(diff_v, out_ref.at[pl.ds((cb + j) * CHK, CHK)])
        return carry
    lax.fori_loop(0, _EF_NCHK, cbody, 0)


def kernel(z, template_x, edge_index, mlp_w1, mlp_b1, mlp_w2, mlp_b2, mlp_w3, mlp_b3, gat1_w, gat1_as, gat1_ad, gat1_b, ln1_g, ln1_b, gat2_w, gat2_as, gat2_ad, gat2_b, ln2_g, ln2_b, gat3_w, gat3_as, gat3_ad, gat3_b):
    f = jnp.float32
    offs = jnp.arange(B, dtype=edge_index.dtype) * N
    src = (edge_index[0][None, :] + offs[:, None]).reshape(-1)
    dst = (edge_index[1][None, :] + offs[:, None]).reshape(-1)
    srcm16 = src.reshape(NSUB, EB // (NSUB * CHK), CHK)
    dstm16 = dst.reshape(NSUB, EB // (NSUB * CHK), CHK)
    srcm32 = src.reshape(NCORES * NSUB, _EF_NCHK, CHK)
    dstm32 = dst.reshape(NCORES * NSUB, _EF_NCHK, CHK)
    srcb = src.reshape(NBATCH, BQ, CHK)
    dstb = dst.reshape(NBATCH, BQ, CHK)
    zeros80 = jnp.zeros((ZT, ZW, HID + 16), f)
    zeros32 = jnp.zeros((ZT, ZW, 32), f)

    w1_heads = gat1_w.reshape(HEADS, HID, GIN)
    w2_heads = gat2_w.reshape(HEADS, HID, F1)
    w3pad = jnp.concatenate([gat3_w, jnp.zeros((3, F1), f)], axis=0).reshape(1, 16, F1)
    as3pad = jnp.concatenate([gat3_as, jnp.zeros((1, 3), f)], axis=1)
    ad3pad = jnp.concatenate([gat3_ad, jnp.zeros((1, 3), f)], axis=1)
    b3pad = jnp.concatenate([gat3_b, jnp.zeros((3,), f)]).reshape(1, 16)

    s = _style_mlp(z, mlp_w1, mlp_b1, mlp_w2, mlp_b2, mlp_w3, mlp_b3)

    x1_t, als1, ald1 = _tc0(template_x, s, w1_heads, gat1_as, gat1_ad)
    fused1 = _edge4(x1_t.reshape(HEADS * NB, HID),
                    als1.T.reshape(-1), ald1.T.reshape(-1),
                    srcm16, dstm16, srcb, dstb, zeros80)

    h1n, x2_t, als2, ald2 = _tcmid(
        fused1.reshape(HEADS, NB, HID + 16), gat1_b.reshape(1, F1),
        ln1_g.reshape(1, F1), ln1_b.reshape(1, F1),
        jnp.zeros((NB, F1), f), w2_heads, gat2_as, gat2_ad, residual=False)
    fused2 = _edge4(x2_t.reshape(HEADS * NB, HID),
                    als2.T.reshape(-1), ald2.T.reshape(-1),
                    srcm16, dstm16, srcb, dstb, zeros80)

    _, x3_t, als3, ald3 = _tcmid(
        fused2.reshape(HEADS, NB, HID + 16), gat2_b.reshape(1, F1),
        ln2_g.reshape(1, F1), ln2_b.reshape(1, F1),
        h1n, w3pad, as3pad.reshape(1, 16), ad3pad.reshape(1, 16),
        residual=True)
    fused3 = _edge1(x3_t.reshape(NB, 16),
                    als3.T.reshape(-1), ald3.T.reshape(-1),
                    srcm32, dstm32, srcb, dstb, zeros32)

    x_gen, p3, p16 = _tc3(fused3.reshape(2, NB, 32), b3pad, template_x)
    ef16 = _ef_kernel(p16, srcm32, dstm32)
    return x_gen, p3, ef16[:, :3]


# speedup vs baseline: 32.6370x; 1.3742x over previous
import functools

import jax
import jax.numpy as jnp
from jax import lax
from jax.experimental import pallas as pl
from jax.experimental.pallas import tpu as pltpu
from jax.experimental.pallas import tpu_sc as plsc

B = 2
N = 10000
E = 160000
NB = B * N
EB = B * E
HEADS = 4
HID = 64
NF = 10
STYLE = 118
GIN = NF + STYLE
F1 = HEADS * HID
NBLK = 400
GRID = NB // NBLK
NPG = N // NBLK

CHK = 80
ECHUNKS = EB // CHK
NCORES = 2
NSUB = 16
ZW = 2000
ZT = NB // ZW


def _leaky(x):
    return jnp.where(x > 0, x, 0.2 * x)


def _mlp_body(z_ref, w1_ref, b1_ref, w2_ref, b2_ref, w3_ref, b3_ref, out_ref):
    h = jnp.dot(z_ref[...], w1_ref[...].T, preferred_element_type=jnp.float32) + b1_ref[...]
    h = _leaky(h)
    h = jnp.dot(h, w2_ref[...].T, preferred_element_type=jnp.float32) + b2_ref[...]
    h = _leaky(h)
    out_ref[...] = jnp.dot(h, w3_ref[...].T, preferred_element_type=jnp.float32) + b3_ref[...]


def _style_mlp(z, w1, b1, w2, b2, w3, b3):
    return pl.pallas_call(
        _mlp_body,
        out_shape=jax.ShapeDtypeStruct((z.shape[0], w3.shape[0]), jnp.float32),
    )(z, w1, b1.reshape(1, -1), w2, b2.reshape(1, -1), w3, b3.reshape(1, -1))


def _tc0_body(xb_ref, s_ref, w1h_ref, as_ref, ad_ref, xt_ref, als_ref, ald_ref):
    g = pl.program_id(0) // NPG
    srow = s_ref[pl.ds(g, 1)]
    h0 = jnp.concatenate(
        [xb_ref[...], jnp.broadcast_to(srow, (NBLK, STYLE))], axis=1)
    als_cols, ald_cols = [], []
    for h in range(HEADS):
        xh = jnp.dot(h0, w1h_ref[h].T, preferred_element_type=jnp.float32)
        xt_ref[h] = xh
        als_cols.append(jnp.sum(xh * as_ref[h][None, :], axis=1, keepdims=True))
        ald_cols.append(jnp.sum(xh * ad_ref[h][None, :], axis=1, keepdims=True))
    als_ref[...] = jnp.concatenate(als_cols, axis=1)
    ald_ref[...] = jnp.concatenate(ald_cols, axis=1)


def _tc0(template_x, s, w1_heads, a_s, a_d):
    return pl.pallas_call(
        _tc0_body,
        grid=(GRID,),
        in_specs=[
            pl.BlockSpec((NBLK, NF), lambda i: (i % NPG, 0)),
            pl.BlockSpec((B, STYLE), lambda i: (0, 0)),
            pl.BlockSpec((HEADS, HID, GIN), lambda i: (0, 0, 0)),
            pl.BlockSpec((HEADS, HID), lambda i: (0, 0)),
            pl.BlockSpec((HEADS, HID), lambda i: (0, 0)),
        ],
        out_specs=[
            pl.BlockSpec((HEADS, NBLK, HID), lambda i: (0, i, 0)),
            pl.BlockSpec((NBLK, HEADS), lambda i: (i, 0)),
            pl.BlockSpec((NBLK, HEADS), lambda i: (i, 0)),
        ],
        out_shape=[
            jax.ShapeDtypeStruct((HEADS, NB, HID), jnp.float32),
            jax.ShapeDtypeStruct((NB, HEADS), jnp.float32),
            jax.ShapeDtypeStruct((NB, HEADS), jnp.float32),
        ],
    )(template_x, s, w1_heads, a_s, a_d)


def _tcmid_body(fused_ref, bias_ref, g_ref, b_ref, res_ref,
                wh_ref, as_ref, ad_ref, hn_ref, xt_ref, als_ref, ald_ref,
                *, residual):
    cols = []
    for h in range(HEADS):
        fh = fused_ref[h]
        cols.append(fh[:, :HID] / (fh[:, HID:HID + 1] + 1e-16))
    hcat = jnp.concatenate(cols, axis=1) + bias_ref[...]
    hcat = _leaky(hcat)
    if residual:
        hcat = hcat + res_ref[...]
    mu = jnp.mean(hcat, axis=-1, keepdims=True)
    var = jnp.mean((hcat - mu) ** 2, axis=-1, keepdims=True)
    hn = g_ref[...] * (hcat - mu) / jnp.sqrt(var + 1e-5) + b_ref[...]
    hn_ref[...] = hn
    nh = xt_ref.shape[0]
    als_cols, ald_cols = [], []
    for h in range(nh):
        xh = jnp.dot(hn, wh_ref[h].T, preferred_element_type=jnp.float32)
        xt_ref[h] = xh
        als_cols.append(jnp.sum(xh * as_ref[h][None, :], axis=1, keepdims=True))
        ald_cols.append(jnp.sum(xh * ad_ref[h][None, :], axis=1, keepdims=True))
    als_ref[...] = jnp.concatenate(als_cols, axis=1)
    ald_ref[...] = jnp.concatenate(ald_cols, axis=1)


def _tcmid(fused, bias, g, b, res, w_heads, a_s, a_d, residual):
    nh, od = w_heads.shape[0], w_heads.shape[1]
    body = functools.partial(_tcmid_body, residual=residual)
    in_specs = [
        pl.BlockSpec((HEADS, NBLK, HID + 16), lambda i: (0, i, 0)),
        pl.BlockSpec((1, F1), lambda i: (0, 0)),
        pl.BlockSpec((1, F1), lambda i: (0, 0)),
        pl.BlockSpec((1, F1), lambda i: (0, 0)),
        pl.BlockSpec((NBLK, F1), lambda i: (i, 0)),
        pl.BlockSpec((nh, od, F1), lambda i: (0, 0, 0)),
        pl.BlockSpec((nh, od), lambda i: (0, 0)),
        pl.BlockSpec((nh, od), lambda i: (0, 0)),
    ]
    return pl.pallas_call(
        body,
        grid=(GRID,),
        in_specs=in_specs,
        out_specs=[
            pl.BlockSpec((NBLK, F1), lambda i: (i, 0)),
            pl.BlockSpec((nh, NBLK, od), lambda i: (0, i, 0)),
            pl.BlockSpec((NBLK, nh), lambda i: (i, 0)),
            pl.BlockSpec((NBLK, nh), lambda i: (i, 0)),
        ],
        out_shape=[
            jax.ShapeDtypeStruct((NB, F1), jnp.float32),
            jax.ShapeDtypeStruct((nh, NB, od), jnp.float32),
            jax.ShapeDtypeStruct((NB, nh), jnp.float32),
            jax.ShapeDtypeStruct((NB, nh), jnp.float32),
        ],
    )(fused, bias, g, b, res, w_heads, a_s, a_d)


def _tc3_body(f_ref, b3_ref, xb_ref, xgen_ref, p3_ref, p16_ref):
    sblk = f_ref[0] + f_ref[1]
    hout = sblk[:, :16] / (sblk[:, 16:17] + 1e-16) + b3_ref[...]
    xb = xb_ref[...]
    p_gen = xb[:, :3] + hout[:, :3]
    xtail = xb[:, 3:] + hout[:, 6:13]
    xgen_ref[...] = jnp.concatenate([p_gen, xtail], axis=1)
    p3_ref[...] = p_gen
    p16_ref[...] = jnp.concatenate(
        [p_gen, jnp.zeros((p_gen.shape[0], 13), jnp.float32)], axis=1)


def _tc3(fused3, b3pad, template_x):
    return pl.pallas_call(
        _tc3_body,
        grid=(GRID,),
        in_specs=[
            pl.BlockSpec((2, NBLK, 32), lambda i: (0, i, 0)),
            pl.BlockSpec((1, 16), lambda i: (0, 0)),
            pl.BlockSpec((NBLK, NF), lambda i: (i % NPG, 0)),
        ],
        out_specs=[
            pl.BlockSpec((NBLK, NF), lambda i: (i, 0)),
            pl.BlockSpec((NBLK, 3), lambda i: (i, 0)),
            pl.BlockSpec((NBLK, 16), lambda i: (i, 0)),
        ],
        out_shape=[
            jax.ShapeDtypeStruct((NB, NF), jnp.float32),
            jax.ShapeDtypeStruct((NB, 3), jnp.float32),
            jax.ShapeDtypeStruct((NB, 16), jnp.float32),
        ],
    )(fused3, b3pad, template_x)


_SC_PARAMS = pltpu.CompilerParams(
    use_tc_tiling_on_sc=False, needs_layout_passes=False)
BQ = 25
BE = BQ * CHK
NBATCH = EB // BE


def _make_ex_sc(heads, edge_split):
    hpc = 1 if edge_split else heads // NCORES
    nchk = ECHUNKS // (NCORES * NSUB) if edge_split else ECHUNKS // NSUB
    slots = NCORES * NSUB if edge_split else heads * NSUB
    mesh = plsc.VectorSubcoreMesh(core_axis_name="c", subcore_axis_name="s")

    @functools.partial(
        pl.kernel,
        mesh=mesh,
        out_type=jax.ShapeDtypeStruct((slots, nchk, CHK), jnp.float32),
        compiler_params=_SC_PARAMS,
        scratch_types=[
            pltpu.VMEM((nchk, CHK), jnp.int32),
            pltpu.VMEM((nchk, CHK), jnp.int32),
            pltpu.VMEM((NB,), jnp.float32),
            pltpu.VMEM((NB,), jnp.float32),
            pltpu.VMEM((nchk, CHK), jnp.float32),
        ],
    )
    def ex_kernel(als_ref, ald_ref, srcm_ref, dstm_ref, out_ref,
                  src_v, dst_v, als_v, ald_v, exb_v):
        c = lax.axis_index("c")
        s = lax.axis_index("s")
        tile = c * NSUB + s if edge_split else s
        pltpu.sync_copy(srcm_ref.at[tile], src_v)
        pltpu.sync_copy(dstm_ref.at[tile], dst_v)
        for slot in range(hpc):
            if edge_split:
                hoff = jnp.int32(0)
                oslot = tile
            else:
                h = c + 2 * slot
                hoff = h * NB
                oslot = h * NSUB + s
            pltpu.sync_copy(als_ref.at[pl.ds(hoff, NB)], als_v)
            pltpu.sync_copy(ald_ref.at[pl.ds(hoff, NB)], ald_v)

            def cbody(j, carry):
                for k in range(CHK // 16):
                    s16 = src_v[j, pl.ds(16 * k, 16)]
                    d16 = dst_v[j, pl.ds(16 * k, 16)]
                    a = (plsc.load_gather(als_v, [s16])
                         + plsc.load_gather(ald_v, [d16]))
                    exb_v[j, pl.ds(16 * k, 16)] = jnp.exp(_leaky(a))
                return carry
            lax.fori_loop(0, nchk, cbody, 0)
            pltpu.sync_copy(exb_v, out_ref.at[oslot])

    return ex_kernel


def _make_scatter_sc(heads, d, edge_split):
    roww = d + 16
    hpc = 1 if edge_split else heads // NCORES
    nbat = NBATCH // (NCORES * NSUB) if edge_split else NBATCH // NSUB
    mesh = plsc.VectorSubcoreMesh(core_axis_name="c", subcore_axis_name="s")
    out_rows = 2 * NB if edge_split else heads * NB

    @functools.partial(
        pl.kernel,
        mesh=mesh,
        out_type=jax.ShapeDtypeStruct((out_rows, roww), jnp.float32),
        compiler_params=_SC_PARAMS,
        scratch_types=[
            pltpu.VMEM((BQ, CHK), jnp.int32),
            pltpu.VMEM((BQ, CHK), jnp.int32),
            pltpu.VMEM((BE,), jnp.float32),
            pltpu.VMEM((2, CHK, d), jnp.float32),
            pltpu.VMEM((2, CHK, roww), jnp.float32),
            pltpu.VMEM_SHARED((NB, roww), jnp.float32),
            pltpu.SemaphoreType.DMA,
            pltpu.SemaphoreType.DMA,
            pltpu.SemaphoreType.DMA,
            pltpu.SemaphoreType.DMA,
        ],
    )
    def scatter_kernel(x_ref, exm_ref, srcb_ref, dstb_ref, z_ref,
                       out_ref, src_v, dst_v, ex_v, rows_v,
                       srow_v, acc, gsem0, gsem1, ssem0, ssem1):
        c = lax.axis_index("c")
        s = lax.axis_index("s")
        tile = c * NSUB + s if edge_split else s
        gsems = [gsem0, gsem1]
        ssems = [ssem0, ssem1]

        for slot in range(hpc):
            if edge_split:
                hoff = None
                out_base = c * NB
                exslot0 = tile * nbat
            else:
                h = c + 2 * slot
                hoff = h * NB
                out_base = h * NB
                exslot0 = (h * NSUB + s) * nbat

            @pl.when(s < ZT)
            def _zero():
                pltpu.sync_copy(z_ref.at[s], acc.at[pl.ds(s * ZW, ZW)])

            plsc.subcore_barrier()

            def bbody(b, carry):
                gb = tile * nbat + b
                pltpu.sync_copy(srcb_ref.at[gb], src_v)
                pltpu.sync_copy(dstb_ref.at[gb], dst_v)
                pltpu.sync_copy(exm_ref.at[exslot0 + b], ex_v)

                if hoff is not None:
                    def gbody(j, gc):
                        for k in range(CHK // 16):
                            src_v[j, pl.ds(16 * k, 16)] = (
                                src_v[j, pl.ds(16 * k, 16)] + hoff)
                        return gc
                    lax.fori_loop(0, BQ, gbody, 0)

                gcps = {}
                scps = {}
                gcps[0] = pltpu.async_copy(
                    x_ref.at[src_v.at[0]], rows_v.at[0], gsems[0])
                for j in range(BQ):
                    buf = j % 2
                    if j + 1 < BQ:
                        gcps[j + 1] = pltpu.async_copy(
                            x_ref.at[src_v.at[j + 1]], rows_v.at[1 - buf],
                            gsems[1 - buf])
                    gcps[j].wait()
                    if j >= 2:
                        scps[j - 2].wait()

                    def rbody(e, rc):
                        erow = jnp.full((16,), 0, jnp.int32) + (j * CHK + e)
                        bc = plsc.load_gather(ex_v, [erow])
                        for f in range(d // 16):
                            srow_v[buf, e, pl.ds(16 * f, 16)] = (
                                rows_v[buf, e, pl.ds(16 * f, 16)] * bc)
                        srow_v[buf, e, pl.ds(d, 16)] = bc
                        return rc
                    lax.fori_loop(0, CHK, rbody, 0)
                    scps[j] = pltpu.async_copy(
                        srow_v.at[buf], acc.at[dst_v.at[j]], ssems[buf],
                        add=True)
                scps[BQ - 2].wait()
                scps[BQ - 1].wait()
                return carry
            lax.fori_loop(0, nbat, bbody, 0)
            plsc.subcore_barrier()

            @pl.when(s < ZT)
            def _writeout():
                pltpu.sync_copy(acc.at[pl.ds(s * ZW, ZW)],
                                out_ref.at[pl.ds(out_base + s * ZW, ZW)])

            plsc.subcore_barrier()

    return scatter_kernel


_ex4 = _make_ex_sc(HEADS, edge_split=False)
_ex1 = _make_ex_sc(1, edge_split=True)
_scat4 = _make_scatter_sc(HEADS, HID, edge_split=False)
_scat1 = _make_scatter_sc(1, 16, edge_split=True)


def _edge4(x_flat, als_f, ald_f, srcm16, dstm16, srcb, dstb, zeros):
    exm = _ex4(als_f, ald_f, srcm16, dstm16)
    return _scat4(x_flat, exm.reshape(-1, BE), srcb, dstb, zeros)


def _edge1(x_flat, als_f, ald_f, srcm32, dstm32, srcb, dstb, zeros):
    exm = _ex1(als_f, ald_f, srcm32, dstm32)
    return _scat1(x_flat, exm.reshape(-1, BE), srcb, dstb, zeros)


_EF_NCHK = ECHUNKS // (NCORES * NSUB)


@functools.partial(
    pl.kernel,
    mesh=plsc.VectorSubcoreMesh(core_axis_name="c", subcore_axis_name="s"),
    out_type=jax.ShapeDtypeStruct((EB, 16), jnp.float32),
    compiler_params=pltpu.CompilerParams(use_tc_tiling_on_sc=False, needs_layout_passes=False),
    scratch_types=[
        pltpu.VMEM((_EF_NCHK, CHK), jnp.int32),
        pltpu.VMEM((_EF_NCHK, CHK), jnp.int32),
        pltpu.VMEM((CHK, 16), jnp.float32),
        pltpu.VMEM((CHK, 16), jnp.float32),
        pltpu.VMEM((CHK, 16), jnp.float32),
        pltpu.SemaphoreType.DMA,
        pltpu.SemaphoreType.DMA,
    ],
)
def _ef_kernel(p_ref, srcm_ref, dstm_ref, out_ref, src_v, dst_v,
               rs_v, rd_v, diff_v, sem_s, sem_d):
    c = lax.axis_index("c")
    s = lax.axis_index("s")
    iota = lax.iota(jnp.int32, 16)
    w = c * NSUB + s
    cb = w * _EF_NCHK
    pltpu.sync_copy(srcm_ref.at[w], src_v)
    pltpu.sync_copy(dstm_ref.at[w], dst_v)

    def cbody(j, carry):
        cps = pltpu.async_copy(p_ref.at[src_v.at[j]], rs_v, sem_s)
        cpd = pltpu.async_copy(p_ref.at[dst_v.at[j]], rd_v, sem_d)
        cps.wait()
        cpd.wait()

        def rbody(e, rc):
            diff_v[e] = rd_v[e] - rs_v[e]
            return rc
        lax.fori_loop(0, CHK, rbody, 0)
        pltpu.sync_copy(diff_v, out_ref.at[pl.ds((cb + j) * CHK, CHK)])
        return carry
    lax.fori_loop(0, _EF_NCHK, cbody, 0)


def kernel(z, template_x, edge_index, mlp_w1, mlp_b1, mlp_w2, mlp_b2, mlp_w3, mlp_b3, gat1_w, gat1_as, gat1_ad, gat1_b, ln1_g, ln1_b, gat2_w, gat2_as, gat2_ad, gat2_b, ln2_g, ln2_b, gat3_w, gat3_as, gat3_ad, gat3_b):
    f = jnp.float32
    offs = jnp.arange(B, dtype=edge_index.dtype) * N
    src = (edge_index[0][None, :] + offs[:, None]).reshape(-1)
    dst = (edge_index[1][None, :] + offs[:, None]).reshape(-1)
    srcm16 = src.reshape(NSUB, EB // (NSUB * CHK), CHK)
    dstm16 = dst.reshape(NSUB, EB // (NSUB * CHK), CHK)
    srcm32 = src.reshape(NCORES * NSUB, _EF_NCHK, CHK)
    dstm32 = dst.reshape(NCORES * NSUB, _EF_NCHK, CHK)
    srcb = src.reshape(NBATCH, BQ, CHK)
    dstb = dst.reshape(NBATCH, BQ, CHK)
    zeros80 = jnp.zeros((ZT, ZW, HID + 16), f)
    zeros32 = jnp.zeros((ZT, ZW, 32), f)

    w1_heads = gat1_w.reshape(HEADS, HID, GIN)
    w2_heads = gat2_w.reshape(HEADS, HID, F1)
    w3pad = jnp.concatenate([gat3_w, jnp.zeros((3, F1), f)], axis=0).reshape(1, 16, F1)
    as3pad = jnp.concatenate([gat3_as, jnp.zeros((1, 3), f)], axis=1)
    ad3pad = jnp.concatenate([gat3_ad, jnp.zeros((1, 3), f)], axis=1)
    b3pad = jnp.concatenate([gat3_b, jnp.zeros((3,), f)]).reshape(1, 16)

    s = _style_mlp(z, mlp_w1, mlp_b1, mlp_w2, mlp_b2, mlp_w3, mlp_b3)

    x1_t, als1, ald1 = _tc0(template_x, s, w1_heads, gat1_as, gat1_ad)
    fused1 = _edge4(x1_t.reshape(HEADS * NB, HID),
                    als1.T.reshape(-1), ald1.T.reshape(-1),
                    srcm16, dstm16, srcb, dstb, zeros80)

    h1n, x2_t, als2, ald2 = _tcmid(
        fused1.reshape(HEADS, NB, HID + 16), gat1_b.reshape(1, F1),
        ln1_g.reshape(1, F1), ln1_b.reshape(1, F1),
        jnp.zeros((NB, F1), f), w2_heads, gat2_as, gat2_ad, residual=False)
    fused2 = _edge4(x2_t.reshape(HEADS * NB, HID),
                    als2.T.reshape(-1), ald2.T.reshape(-1),
                    srcm16, dstm16, srcb, dstb, zeros80)

    _, x3_t, als3, ald3 = _tcmid(
        fused2.reshape(HEADS, NB, HID + 16), gat2_b.reshape(1, F1),
        ln2_g.reshape(1, F1), ln2_b.reshape(1, F1),
        h1n, w3pad, as3pad.reshape(1, 16), ad3pad.reshape(1, 16),
        residual=True)
    fused3 = _edge1(x3_t.reshape(NB, 16),
                    als3.T.reshape(-1), ald3.T.reshape(-1),
                    srcm32, dstm32, srcb, dstb, zeros32)

    x_gen, p3, p16 = _tc3(fused3.reshape(2, NB, 32), b3pad, template_x)
    ef16 = _ef_kernel(p16, srcm32, dstm32)
    return x_gen, p3, ef16[:, :3]


# speedup vs baseline: 33.1065x; 1.0144x over previous
import functools

import jax
import jax.numpy as jnp
from jax import lax
from jax.experimental import pallas as pl
from jax.experimental.pallas import tpu as pltpu
from jax.experimental.pallas import tpu_sc as plsc

B = 2
N = 10000
E = 160000
NB = B * N
EB = B * E
HEADS = 4
HID = 64
NF = 10
STYLE = 118
GIN = NF + STYLE
F1 = HEADS * HID
NBLK = 400
GRID = NB // NBLK
NPG = N // NBLK

CHK = 80
ECHUNKS = EB // CHK
NCORES = 2
NSUB = 16
ZW = 2000
ZT = NB // ZW


def _leaky(x):
    return jnp.where(x > 0, x, 0.2 * x)


def _mlp_body(z_ref, w1_ref, b1_ref, w2_ref, b2_ref, w3_ref, b3_ref, out_ref):
    h = jnp.dot(z_ref[...], w1_ref[...].T, preferred_element_type=jnp.float32) + b1_ref[...]
    h = _leaky(h)
    h = jnp.dot(h, w2_ref[...].T, preferred_element_type=jnp.float32) + b2_ref[...]
    h = _leaky(h)
    out_ref[...] = jnp.dot(h, w3_ref[...].T, preferred_element_type=jnp.float32) + b3_ref[...]


def _style_mlp(z, w1, b1, w2, b2, w3, b3):
    return pl.pallas_call(
        _mlp_body,
        out_shape=jax.ShapeDtypeStruct((z.shape[0], w3.shape[0]), jnp.float32),
    )(z, w1, b1.reshape(1, -1), w2, b2.reshape(1, -1), w3, b3.reshape(1, -1))


def _tc0_body(xb_ref, s_ref, w1h_ref, as_ref, ad_ref, xt_ref, als_ref, ald_ref):
    g = pl.program_id(0) // NPG
    srow = s_ref[pl.ds(g, 1)]
    h0 = jnp.concatenate(
        [xb_ref[...], jnp.broadcast_to(srow, (NBLK, STYLE))], axis=1)
    als_cols, ald_cols = [], []
    for h in range(HEADS):
        xh = jnp.dot(h0, w1h_ref[h].T, preferred_element_type=jnp.float32)
        xt_ref[h] = xh
        als_cols.append(jnp.sum(xh * as_ref[h][None, :], axis=1, keepdims=True))
        ald_cols.append(jnp.sum(xh * ad_ref[h][None, :], axis=1, keepdims=True))
    als_ref[...] = jnp.concatenate(als_cols, axis=1)
    ald_ref[...] = jnp.concatenate(ald_cols, axis=1)


def _tc0(template_x, s, w1_heads, a_s, a_d):
    return pl.pallas_call(
        _tc0_body,
        grid=(GRID,),
        in_specs=[
            pl.BlockSpec((NBLK, NF), lambda i: (i % NPG, 0)),
            pl.BlockSpec((B, STYLE), lambda i: (0, 0)),
            pl.BlockSpec((HEADS, HID, GIN), lambda i: (0, 0, 0)),
            pl.BlockSpec((HEADS, HID), lambda i: (0, 0)),
            pl.BlockSpec((HEADS, HID), lambda i: (0, 0)),
        ],
        out_specs=[
            pl.BlockSpec((HEADS, NBLK, HID), lambda i: (0, i, 0)),
            pl.BlockSpec((NBLK, HEADS), lambda i: (i, 0)),
            pl.BlockSpec((NBLK, HEADS), lambda i: (i, 0)),
        ],
        out_shape=[
            jax.ShapeDtypeStruct((HEADS, NB, HID), jnp.float32),
            jax.ShapeDtypeStruct((NB, HEADS), jnp.float32),
            jax.ShapeDtypeStruct((NB, HEADS), jnp.float32),
        ],
    )(template_x, s, w1_heads, a_s, a_d)


def _tcmid_body(fused_ref, bias_ref, g_ref, b_ref, res_ref,
                wh_ref, as_ref, ad_ref, hn_ref, xt_ref, als_ref, ald_ref,
                *, residual):
    cols = []
    for h in range(HEADS):
        fh = fused_ref[h]
        cols.append(fh[:, :HID] / (fh[:, HID:HID + 1] + 1e-16))
    hcat = jnp.concatenate(cols, axis=1) + bias_ref[...]
    hcat = _leaky(hcat)
    if residual:
        hcat = hcat + res_ref[...]
    mu = jnp.mean(hcat, axis=-1, keepdims=True)
    var = jnp.mean((hcat - mu) ** 2, axis=-1, keepdims=True)
    hn = g_ref[...] * (hcat - mu) / jnp.sqrt(var + 1e-5) + b_ref[...]
    hn_ref[...] = hn
    nh = xt_ref.shape[0]
    als_cols, ald_cols = [], []
    for h in range(nh):
        xh = jnp.dot(hn, wh_ref[h].T, preferred_element_type=jnp.float32)
        xt_ref[h] = xh
        als_cols.append(jnp.sum(xh * as_ref[h][None, :], axis=1, keepdims=True))
        ald_cols.append(jnp.sum(xh * ad_ref[h][None, :], axis=1, keepdims=True))
    als_ref[...] = jnp.concatenate(als_cols, axis=1)
    ald_ref[...] = jnp.concatenate(ald_cols, axis=1)


def _tcmid(fused, bias, g, b, res, w_heads, a_s, a_d, residual):
    nh, od = w_heads.shape[0], w_heads.shape[1]
    body = functools.partial(_tcmid_body, residual=residual)
    in_specs = [
        pl.BlockSpec((HEADS, NBLK, HID + 16), lambda i: (0, i, 0)),
        pl.BlockSpec((1, F1), lambda i: (0, 0)),
        pl.BlockSpec((1, F1), lambda i: (0, 0)),
        pl.BlockSpec((1, F1), lambda i: (0, 0)),
        pl.BlockSpec((NBLK, F1), lambda i: (i, 0)),
        pl.BlockSpec((nh, od, F1), lambda i: (0, 0, 0)),
        pl.BlockSpec((nh, od), lambda i: (0, 0)),
        pl.BlockSpec((nh, od), lambda i: (0, 0)),
    ]
    return pl.pallas_call(
        body,
        grid=(GRID,),
        in_specs=in_specs,
        out_specs=[
            pl.BlockSpec((NBLK, F1), lambda i: (i, 0)),
            pl.BlockSpec((nh, NBLK, od), lambda i: (0, i, 0)),
            pl.BlockSpec((NBLK, nh), lambda i: (i, 0)),
            pl.BlockSpec((NBLK, nh), lambda i: (i, 0)),
        ],
        out_shape=[
            jax.ShapeDtypeStruct((NB, F1), jnp.float32),
            jax.ShapeDtypeStruct((nh, NB, od), jnp.float32),
            jax.ShapeDtypeStruct((NB, nh), jnp.float32),
            jax.ShapeDtypeStruct((NB, nh), jnp.float32),
        ],
    )(fused, bias, g, b, res, w_heads, a_s, a_d)


def _tc3_body(f_ref, b3_ref, xb_ref, xgen_ref, p3_ref, p16_ref):
    sblk = f_ref[0] + f_ref[1]
    hout = sblk[:, :16] / (sblk[:, 16:17] + 1e-16) + b3_ref[...]
    xb = xb_ref[...]
    p_gen = xb[:, :3] + hout[:, :3]
    xtail = xb[:, 3:] + hout[:, 6:13]
    xgen_ref[...] = jnp.concatenate([p_gen, xtail], axis=1)
    p3_ref[...] = p_gen
    p16_ref[...] = jnp.concatenate(
        [p_gen, jnp.zeros((p_gen.shape[0], 13), jnp.float32)], axis=1)


def _tc3(fused3, b3pad, template_x):
    return pl.pallas_call(
        _tc3_body,
        grid=(GRID,),
        in_specs=[
            pl.BlockSpec((2, NBLK, 32), lambda i: (0, i, 0)),
            pl.BlockSpec((1, 16), lambda i: (0, 0)),
            pl.BlockSpec((NBLK, NF), lambda i: (i % NPG, 0)),
        ],
        out_specs=[
            pl.BlockSpec((NBLK, NF), lambda i: (i, 0)),
            pl.BlockSpec((NBLK, 3), lambda i: (i, 0)),
            pl.BlockSpec((NBLK, 16), lambda i: (i, 0)),
        ],
        out_shape=[
            jax.ShapeDtypeStruct((NB, NF), jnp.float32),
            jax.ShapeDtypeStruct((NB, 3), jnp.float32),
            jax.ShapeDtypeStruct((NB, 16), jnp.float32),
        ],
    )(fused3, b3pad, template_x)


_SC_PARAMS = pltpu.CompilerParams(
    use_tc_tiling_on_sc=False, needs_layout_passes=False)
BQ = 25
BE = BQ * CHK
NBATCH = EB // BE


def _make_ex_sc(heads, edge_split):
    hpc = 1 if edge_split else heads // NCORES
    nchk = ECHUNKS // (NCORES * NSUB) if edge_split else ECHUNKS // NSUB
    slots = NCORES * NSUB if edge_split else heads * NSUB
    mesh = plsc.VectorSubcoreMesh(core_axis_name="c", subcore_axis_name="s")

    @functools.partial(
        pl.kernel,
        mesh=mesh,
        out_type=jax.ShapeDtypeStruct((slots, nchk, CHK), jnp.float32),
        compiler_params=_SC_PARAMS,
        scratch_types=[
            pltpu.VMEM((nchk, CHK), jnp.int32),
            pltpu.VMEM((nchk, CHK), jnp.int32),
            pltpu.VMEM((NB,), jnp.float32),
            pltpu.VMEM((NB,), jnp.float32),
            pltpu.VMEM((nchk, CHK), jnp.float32),
        ],
    )
    def ex_kernel(als_ref, ald_ref, srcm_ref, dstm_ref, out_ref,
                  src_v, dst_v, als_v, ald_v, exb_v):
        c = lax.axis_index("c")
        s = lax.axis_index("s")
        tile = c * NSUB + s if edge_split else s
        pltpu.sync_copy(srcm_ref.at[tile], src_v)
        pltpu.sync_copy(dstm_ref.at[tile], dst_v)
        for slot in range(hpc):
            if edge_split:
                hoff = jnp.int32(0)
                oslot = tile
            else:
                h = c + 2 * slot
                hoff = h * NB
                oslot = h * NSUB + s
            pltpu.sync_copy(als_ref.at[pl.ds(hoff, NB)], als_v)
            pltpu.sync_copy(ald_ref.at[pl.ds(hoff, NB)], ald_v)

            def cbody(j, carry):
                for k in range(CHK // 16):
                    s16 = src_v[j, pl.ds(16 * k, 16)]
                    d16 = dst_v[j, pl.ds(16 * k, 16)]
                    a = (plsc.load_gather(als_v, [s16])
                         + plsc.load_gather(ald_v, [d16]))
                    exb_v[j, pl.ds(16 * k, 16)] = jnp.exp(_leaky(a))
                return carry
            lax.fori_loop(0, nchk, cbody, 0)
            pltpu.sync_copy(exb_v, out_ref.at[oslot])

    return ex_kernel


def _make_scatter_sc(heads, d, edge_split):
    roww = d + 16
    hpc = 1 if edge_split else heads // NCORES
    nbat = NBATCH // (NCORES * NSUB) if edge_split else NBATCH // NSUB
    mesh = plsc.VectorSubcoreMesh(core_axis_name="c", subcore_axis_name="s")
    out_rows = 2 * NB if edge_split else heads * NB

    @functools.partial(
        pl.kernel,
        mesh=mesh,
        out_type=jax.ShapeDtypeStruct((out_rows, roww), jnp.float32),
        compiler_params=_SC_PARAMS,
        scratch_types=[
            pltpu.VMEM((BQ, CHK), jnp.int32),
            pltpu.VMEM((BQ, CHK), jnp.int32),
            pltpu.VMEM((BE,), jnp.float32),
            pltpu.VMEM((2, CHK, d), jnp.float32),
            pltpu.VMEM((2, CHK, roww), jnp.float32),
            pltpu.VMEM_SHARED((NB, roww), jnp.float32),
            pltpu.SemaphoreType.DMA,
            pltpu.SemaphoreType.DMA,
            pltpu.SemaphoreType.DMA,
            pltpu.SemaphoreType.DMA,
        ],
    )
    def scatter_kernel(x_ref, exm_ref, srcb_ref, dstb_ref, z_ref,
                       out_ref, src_v, dst_v, ex_v, rows_v,
                       srow_v, acc, gsem0, gsem1, ssem0, ssem1):
        c = lax.axis_index("c")
        s = lax.axis_index("s")
        tile = c * NSUB + s if edge_split else s
        gsems = [gsem0, gsem1]
        ssems = [ssem0, ssem1]

        for slot in range(hpc):
            if edge_split:
                hoff = None
                out_base = c * NB
                exslot0 = tile * nbat
            else:
                h = c + 2 * slot
                hoff = h * NB
                out_base = h * NB
                exslot0 = (h * NSUB + s) * nbat

            @pl.when(s < ZT)
            def _zero():
                pltpu.sync_copy(z_ref.at[s], acc.at[pl.ds(s * ZW, ZW)])

            plsc.subcore_barrier()

            def bbody(b, carry):
                gb = tile * nbat + b
                pltpu.sync_copy(srcb_ref.at[gb], src_v)
                pltpu.sync_copy(dstb_ref.at[gb], dst_v)
                pltpu.sync_copy(exm_ref.at[exslot0 + b], ex_v)

                if hoff is not None:
                    def gbody(j, gc):
                        for k in range(CHK // 16):
                            src_v[j, pl.ds(16 * k, 16)] = (
                                src_v[j, pl.ds(16 * k, 16)] + hoff)
                        return gc
                    lax.fori_loop(0, BQ, gbody, 0)

                gcps = {}
                scps = {}
                gcps[0] = pltpu.async_copy(
                    x_ref.at[src_v.at[0]], rows_v.at[0], gsems[0])
                for j in range(BQ):
                    buf = j % 2
                    if j + 1 < BQ:
                        gcps[j + 1] = pltpu.async_copy(
                            x_ref.at[src_v.at[j + 1]], rows_v.at[1 - buf],
                            gsems[1 - buf])
                    gcps[j].wait()
                    if j >= 2:
                        scps[j - 2].wait()

                    def rbody(e, rc):
                        erow = jnp.full((16,), 0, jnp.int32) + (j * CHK + e)
                        bc = plsc.load_gather(ex_v, [erow])
                        for f in range(d // 16):
                            srow_v[buf, e, pl.ds(16 * f, 16)] = (
                                rows_v[buf, e, pl.ds(16 * f, 16)] * bc)
                        srow_v[buf, e, pl.ds(d, 16)] = bc
                        return rc
                    lax.fori_loop(0, CHK, rbody, 0, unroll=2)
                    scps[j] = pltpu.async_copy(
                        srow_v.at[buf], acc.at[dst_v.at[j]], ssems[buf],
                        add=True)
                scps[BQ - 2].wait()
                scps[BQ - 1].wait()
                return carry
            lax.fori_loop(0, nbat, bbody, 0)
            plsc.subcore_barrier()

            @pl.when(s < ZT)
            def _writeout():
                pltpu.sync_copy(acc.at[pl.ds(s * ZW, ZW)],
                                out_ref.at[pl.ds(out_base + s * ZW, ZW)])

            plsc.subcore_barrier()

    return scatter_kernel


_ex4 = _make_ex_sc(HEADS, edge_split=False)
_ex1 = _make_ex_sc(1, edge_split=True)
_scat4 = _make_scatter_sc(HEADS, HID, edge_split=False)
_scat1 = _make_scatter_sc(1, 16, edge_split=True)


def _edge4(x_flat, als_f, ald_f, srcm16, dstm16, srcb, dstb, zeros):
    exm = _ex4(als_f, ald_f, srcm16, dstm16)
    return _scat4(x_flat, exm.reshape(-1, BE), srcb, dstb, zeros)


def _edge1(x_flat, als_f, ald_f, srcm32, dstm32, srcb, dstb, zeros):
    exm = _ex1(als_f, ald_f, srcm32, dstm32)
    return _scat1(x_flat, exm.reshape(-1, BE), srcb, dstb, zeros)


_EF_NCHK = ECHUNKS // (NCORES * NSUB)


@functools.partial(
    pl.kernel,
    mesh=plsc.VectorSubcoreMesh(core_axis_name="c", subcore_axis_name="s"),
    out_type=jax.ShapeDtypeStruct((EB, 16), jnp.float32),
    compiler_params=pltpu.CompilerParams(use_tc_tiling_on_sc=False, needs_layout_passes=False),
    scratch_types=[
        pltpu.VMEM((_EF_NCHK, CHK), jnp.int32),
        pltpu.VMEM((_EF_NCHK, CHK), jnp.int32),
        pltpu.VMEM((CHK, 16), jnp.float32),
        pltpu.VMEM((CHK, 16), jnp.float32),
        pltpu.VMEM((CHK, 16), jnp.float32),
        pltpu.SemaphoreType.DMA,
        pltpu.SemaphoreType.DMA,
    ],
)
def _ef_kernel(p_ref, srcm_ref, dstm_ref, out_ref, src_v, dst_v,
               rs_v, rd_v, diff_v, sem_s, sem_d):
    c = lax.axis_index("c")
    s = lax.axis_index("s")
    iota = lax.iota(jnp.int32, 16)
    w = c * NSUB + s
    cb = w * _EF_NCHK
    pltpu.sync_copy(srcm_ref.at[w], src_v)
    pltpu.sync_copy(dstm_ref.at[w], dst_v)

    def cbody(j, carry):
        cps = pltpu.async_copy(p_ref.at[src_v.at[j]], rs_v, sem_s)
        cpd = pltpu.async_copy(p_ref.at[dst_v.at[j]], rd_v, sem_d)
        cps.wait()
        cpd.wait()

        def rbody(e, rc):
            diff_v[e] = rd_v[e] - rs_v[e]
            return rc
        lax.fori_loop(0, CHK, rbody, 0)
        pltpu.sync_copy(diff_v, out_ref.at[pl.ds((cb + j) * CHK, CHK)])
        return carry
    lax.fori_loop(0, _EF_NCHK, cbody, 0)


def kernel(z, template_x, edge_index, mlp_w1, mlp_b1, mlp_w2, mlp_b2, mlp_w3, mlp_b3, gat1_w, gat1_as, gat1_ad, gat1_b, ln1_g, ln1_b, gat2_w, gat2_as, gat2_ad, gat2_b, ln2_g, ln2_b, gat3_w, gat3_as, gat3_ad, gat3_b):
    f = jnp.float32
    offs = jnp.arange(B, dtype=edge_index.dtype) * N
    src = (edge_index[0][None, :] + offs[:, None]).reshape(-1)
    dst = (edge_index[1][None, :] + offs[:, None]).reshape(-1)
    srcm16 = src.reshape(NSUB, EB // (NSUB * CHK), CHK)
    dstm16 = dst.reshape(NSUB, EB // (NSUB * CHK), CHK)
    srcm32 = src.reshape(NCORES * NSUB, _EF_NCHK, CHK)
    dstm32 = dst.reshape(NCORES * NSUB, _EF_NCHK, CHK)
    srcb = src.reshape(NBATCH, BQ, CHK)
    dstb = dst.reshape(NBATCH, BQ, CHK)
    zeros80 = jnp.zeros((ZT, ZW, HID + 16), f)
    zeros32 = jnp.zeros((ZT, ZW, 32), f)

    w1_heads = gat1_w.reshape(HEADS, HID, GIN)
    w2_heads = gat2_w.reshape(HEADS, HID, F1)
    w3pad = jnp.concatenate([gat3_w, jnp.zeros((3, F1), f)], axis=0).reshape(1, 16, F1)
    as3pad = jnp.concatenate([gat3_as, jnp.zeros((1, 3), f)], axis=1)
    ad3pad = jnp.concatenate([gat3_ad, jnp.zeros((1, 3), f)], axis=1)
    b3pad = jnp.concatenate([gat3_b, jnp.zeros((3,), f)]).reshape(1, 16)

    s = _style_mlp(z, mlp_w1, mlp_b1, mlp_w2, mlp_b2, mlp_w3, mlp_b3)

    x1_t, als1, ald1 = _tc0(template_x, s, w1_heads, gat1_as, gat1_ad)
    fused1 = _edge4(x1_t.reshape(HEADS * NB, HID),
                    als1.T.reshape(-1), ald1.T.reshape(-1),
                    srcm16, dstm16, srcb, dstb, zeros80)

    h1n, x2_t, als2, ald2 = _tcmid(
        fused1.reshape(HEADS, NB, HID + 16), gat1_b.reshape(1, F1),
        ln1_g.reshape(1, F1), ln1_b.reshape(1, F1),
        jnp.zeros((NB, F1), f), w2_heads, gat2_as, gat2_ad, residual=False)
    fused2 = _edge4(x2_t.reshape(HEADS * NB, HID),
                    als2.T.reshape(-1), ald2.T.reshape(-1),
                    srcm16, dstm16, srcb, dstb, zeros80)

    _, x3_t, als3, ald3 = _tcmid(
        fused2.reshape(HEADS, NB, HID + 16), gat2_b.reshape(1, F1),
        ln2_g.reshape(1, F1), ln2_b.reshape(1, F1),
        h1n, w3pad, as3pad.reshape(1, 16), ad3pad.reshape(1, 16),
        residual=True)
    fused3 = _edge1(x3_t.reshape(NB, 16),
                    als3.T.reshape(-1), ald3.T.reshape(-1),
                    srcm32, dstm32, srcb, dstb, zeros32)

    x_gen, p3, p16 = _tc3(fused3.reshape(2, NB, 32), b3pad, template_x)
    ef16 = _ef_kernel(p16, srcm32, dstm32)
    return x_gen, p3, ef16[:, :3]


# speedup vs baseline: 33.4358x; 1.0099x over previous
import functools

import jax
import jax.numpy as jnp
from jax import lax
from jax.experimental import pallas as pl
from jax.experimental.pallas import tpu as pltpu
from jax.experimental.pallas import tpu_sc as plsc

B = 2
N = 10000
E = 160000
NB = B * N
EB = B * E
HEADS = 4
HID = 64
NF = 10
STYLE = 118
GIN = NF + STYLE
F1 = HEADS * HID
NBLK = 400
GRID = NB // NBLK
NPG = N // NBLK

CHK = 80
ECHUNKS = EB // CHK
NCORES = 2
NSUB = 16
ZW = 2000
ZT = NB // ZW


def _leaky(x):
    return jnp.where(x > 0, x, 0.2 * x)


def _mlp_body(z_ref, w1_ref, b1_ref, w2_ref, b2_ref, w3_ref, b3_ref, out_ref):
    h = jnp.dot(z_ref[...], w1_ref[...].T, preferred_element_type=jnp.float32) + b1_ref[...]
    h = _leaky(h)
    h = jnp.dot(h, w2_ref[...].T, preferred_element_type=jnp.float32) + b2_ref[...]
    h = _leaky(h)
    out_ref[...] = jnp.dot(h, w3_ref[...].T, preferred_element_type=jnp.float32) + b3_ref[...]


def _style_mlp(z, w1, b1, w2, b2, w3, b3):
    return pl.pallas_call(
        _mlp_body,
        out_shape=jax.ShapeDtypeStruct((z.shape[0], w3.shape[0]), jnp.float32),
    )(z, w1, b1.reshape(1, -1), w2, b2.reshape(1, -1), w3, b3.reshape(1, -1))


def _tc0_body(xb_ref, s_ref, w1h_ref, as_ref, ad_ref, xt_ref, als_ref, ald_ref):
    g = pl.program_id(0) // NPG
    srow = s_ref[pl.ds(g, 1)]
    h0 = jnp.concatenate(
        [xb_ref[...], jnp.broadcast_to(srow, (NBLK, STYLE))], axis=1)
    als_cols, ald_cols = [], []
    for h in range(HEADS):
        xh = jnp.dot(h0, w1h_ref[h].T, preferred_element_type=jnp.float32)
        xt_ref[h] = xh
        als_cols.append(jnp.sum(xh * as_ref[h][None, :], axis=1, keepdims=True))
        ald_cols.append(jnp.sum(xh * ad_ref[h][None, :], axis=1, keepdims=True))
    als_ref[...] = jnp.concatenate(als_cols, axis=1)
    ald_ref[...] = jnp.concatenate(ald_cols, axis=1)


def _tc0(template_x, s, w1_heads, a_s, a_d):
    return pl.pallas_call(
        _tc0_body,
        grid=(GRID,),
        in_specs=[
            pl.BlockSpec((NBLK, NF), lambda i: (i % NPG, 0)),
            pl.BlockSpec((B, STYLE), lambda i: (0, 0)),
            pl.BlockSpec((HEADS, HID, GIN), lambda i: (0, 0, 0)),
            pl.BlockSpec((HEADS, HID), lambda i: (0, 0)),
            pl.BlockSpec((HEADS, HID), lambda i: (0, 0)),
        ],
        out_specs=[
            pl.BlockSpec((HEADS, NBLK, HID), lambda i: (0, i, 0)),
            pl.BlockSpec((NBLK, HEADS), lambda i: (i, 0)),
            pl.BlockSpec((NBLK, HEADS), lambda i: (i, 0)),
        ],
        out_shape=[
            jax.ShapeDtypeStruct((HEADS, NB, HID), jnp.float32),
            jax.ShapeDtypeStruct((NB, HEADS), jnp.float32),
            jax.ShapeDtypeStruct((NB, HEADS), jnp.float32),
        ],
    )(template_x, s, w1_heads, a_s, a_d)


def _tcmid_body(fused_ref, bias_ref, g_ref, b_ref, res_ref,
                wh_ref, as_ref, ad_ref, hn_ref, xt_ref, als_ref, ald_ref,
                *, residual):
    cols = []
    for h in range(HEADS):
        fh = fused_ref[h]
        cols.append(fh[:, :HID] / (fh[:, HID:HID + 1] + 1e-16))
    hcat = jnp.concatenate(cols, axis=1) + bias_ref[...]
    hcat = _leaky(hcat)
    if residual:
        hcat = hcat + res_ref[...]
    mu = jnp.mean(hcat, axis=-1, keepdims=True)
    var = jnp.mean((hcat - mu) ** 2, axis=-1, keepdims=True)
    hn = g_ref[...] * (hcat - mu) / jnp.sqrt(var + 1e-5) + b_ref[...]
    hn_ref[...] = hn
    nh = xt_ref.shape[0]
    als_cols, ald_cols = [], []
    for h in range(nh):
        xh = jnp.dot(hn, wh_ref[h].T, preferred_element_type=jnp.float32)
        xt_ref[h] = xh
        als_cols.append(jnp.sum(xh * as_ref[h][None, :], axis=1, keepdims=True))
        ald_cols.append(jnp.sum(xh * ad_ref[h][None, :], axis=1, keepdims=True))
    als_ref[...] = jnp.concatenate(als_cols, axis=1)
    ald_ref[...] = jnp.concatenate(ald_cols, axis=1)


def _tcmid(fused, bias, g, b, res, w_heads, a_s, a_d, residual):
    nh, od = w_heads.shape[0], w_heads.shape[1]
    body = functools.partial(_tcmid_body, residual=residual)
    in_specs = [
        pl.BlockSpec((HEADS, NBLK, HID + 16), lambda i: (0, i, 0)),
        pl.BlockSpec((1, F1), lambda i: (0, 0)),
        pl.BlockSpec((1, F1), lambda i: (0, 0)),
        pl.BlockSpec((1, F1), lambda i: (0, 0)),
        pl.BlockSpec((NBLK, F1), lambda i: (i, 0)),
        pl.BlockSpec((nh, od, F1), lambda i: (0, 0, 0)),
        pl.BlockSpec((nh, od), lambda i: (0, 0)),
        pl.BlockSpec((nh, od), lambda i: (0, 0)),
    ]
    return pl.pallas_call(
        body,
        grid=(GRID,),
        in_specs=in_specs,
        out_specs=[
            pl.BlockSpec((NBLK, F1), lambda i: (i, 0)),
            pl.BlockSpec((nh, NBLK, od), lambda i: (0, i, 0)),
            pl.BlockSpec((NBLK, nh), lambda i: (i, 0)),
            pl.BlockSpec((NBLK, nh), lambda i: (i, 0)),
        ],
        out_shape=[
            jax.ShapeDtypeStruct((NB, F1), jnp.float32),
            jax.ShapeDtypeStruct((nh, NB, od), jnp.float32),
            jax.ShapeDtypeStruct((NB, nh), jnp.float32),
            jax.ShapeDtypeStruct((NB, nh), jnp.float32),
        ],
    )(fused, bias, g, b, res, w_heads, a_s, a_d)


def _tc3_body(f_ref, b3_ref, xb_ref, xgen_ref, p3_ref, p16_ref):
    sblk = f_ref[0] + f_ref[1]
    hout = sblk[:, :16] / (sblk[:, 16:17] + 1e-16) + b3_ref[...]
    xb = xb_ref[...]
    p_gen = xb[:, :3] + hout[:, :3]
    xtail = xb[:, 3:] + hout[:, 6:13]
    xgen_ref[...] = jnp.concatenate([p_gen, xtail], axis=1)
    p3_ref[...] = p_gen
    p16_ref[...] = jnp.concatenate(
        [p_gen, jnp.zeros((p_gen.shape[0], 13), jnp.float32)], axis=1)


def _tc3(fused3, b3pad, template_x):
    return pl.pallas_call(
        _tc3_body,
        grid=(GRID,),
        in_specs=[
            pl.BlockSpec((2, NBLK, 32), lambda i: (0, i, 0)),
            pl.BlockSpec((1, 16), lambda i: (0, 0)),
            pl.BlockSpec((NBLK, NF), lambda i: (i % NPG, 0)),
        ],
        out_specs=[
            pl.BlockSpec((NBLK, NF), lambda i: (i, 0)),
            pl.BlockSpec((NBLK, 3), lambda i: (i, 0)),
            pl.BlockSpec((NBLK, 16), lambda i: (i, 0)),
        ],
        out_shape=[
            jax.ShapeDtypeStruct((NB, NF), jnp.float32),
            jax.ShapeDtypeStruct((NB, 3), jnp.float32),
            jax.ShapeDtypeStruct((NB, 16), jnp.float32),
        ],
    )(fused3, b3pad, template_x)


_SC_PARAMS = pltpu.CompilerParams(
    use_tc_tiling_on_sc=False, needs_layout_passes=False)
BQ = 25
BE = BQ * CHK
NBATCH = EB // BE


def _make_ex_sc(heads, edge_split):
    hpc = 1 if edge_split else heads // NCORES
    nchk = ECHUNKS // (NCORES * NSUB) if edge_split else ECHUNKS // NSUB
    slots = NCORES * NSUB if edge_split else heads * NSUB
    mesh = plsc.VectorSubcoreMesh(core_axis_name="c", subcore_axis_name="s")

    @functools.partial(
        pl.kernel,
        mesh=mesh,
        out_type=jax.ShapeDtypeStruct((slots, nchk, CHK), jnp.float32),
        compiler_params=_SC_PARAMS,
        scratch_types=[
            pltpu.VMEM((nchk, CHK), jnp.int32),
            pltpu.VMEM((nchk, CHK), jnp.int32),
            pltpu.VMEM((NB,), jnp.float32),
            pltpu.VMEM((NB,), jnp.float32),
            pltpu.VMEM((nchk, CHK), jnp.float32),
        ],
    )
    def ex_kernel(als_ref, ald_ref, srcm_ref, dstm_ref, out_ref,
                  src_v, dst_v, als_v, ald_v, exb_v):
        c = lax.axis_index("c")
        s = lax.axis_index("s")
        tile = c * NSUB + s if edge_split else s
        pltpu.sync_copy(srcm_ref.at[tile], src_v)
        pltpu.sync_copy(dstm_ref.at[tile], dst_v)
        for slot in range(hpc):
            if edge_split:
                hoff = jnp.int32(0)
                oslot = tile
            else:
                h = c + 2 * slot
                hoff = h * NB
                oslot = h * NSUB + s
            pltpu.sync_copy(als_ref.at[pl.ds(hoff, NB)], als_v)
            pltpu.sync_copy(ald_ref.at[pl.ds(hoff, NB)], ald_v)

            def cbody(j, carry):
                for k in range(CHK // 16):
                    s16 = src_v[j, pl.ds(16 * k, 16)]
                    d16 = dst_v[j, pl.ds(16 * k, 16)]
                    a = (plsc.load_gather(als_v, [s16])
                         + plsc.load_gather(ald_v, [d16]))
                    exb_v[j, pl.ds(16 * k, 16)] = jnp.exp(_leaky(a))
                return carry
            lax.fori_loop(0, nchk, cbody, 0)
            pltpu.sync_copy(exb_v, out_ref.at[oslot])

    return ex_kernel


def _make_scatter_sc(heads, d, edge_split):
    roww = d + 16
    hpc = 1 if edge_split else heads // NCORES
    nbat = NBATCH // (NCORES * NSUB) if edge_split else NBATCH // NSUB
    mesh = plsc.VectorSubcoreMesh(core_axis_name="c", subcore_axis_name="s")
    out_rows = 2 * NB if edge_split else heads * NB

    @functools.partial(
        pl.kernel,
        mesh=mesh,
        out_type=jax.ShapeDtypeStruct((out_rows, roww), jnp.float32),
        compiler_params=_SC_PARAMS,
        scratch_types=[
            pltpu.VMEM((BQ, CHK), jnp.int32),
            pltpu.VMEM((BQ, CHK), jnp.int32),
            pltpu.VMEM((BE,), jnp.float32),
            pltpu.VMEM((2, CHK, d), jnp.float32),
            pltpu.VMEM((2, CHK, roww), jnp.float32),
            pltpu.VMEM_SHARED((NB, roww), jnp.float32),
            pltpu.SemaphoreType.DMA,
            pltpu.SemaphoreType.DMA,
            pltpu.SemaphoreType.DMA,
            pltpu.SemaphoreType.DMA,
        ],
    )
    def scatter_kernel(x_ref, exm_ref, srcb_ref, dstb_ref, z_ref,
                       out_ref, src_v, dst_v, ex_v, rows_v,
                       srow_v, acc, gsem0, gsem1, ssem0, ssem1):
        c = lax.axis_index("c")
        s = lax.axis_index("s")
        tile = c * NSUB + s if edge_split else s
        gsems = [gsem0, gsem1]
        ssems = [ssem0, ssem1]

        for slot in range(hpc):
            if edge_split:
                hoff = None
                out_base = c * NB
                exslot0 = tile * nbat
            else:
                h = c + 2 * slot
                hoff = h * NB
                out_base = h * NB
                exslot0 = (h * NSUB + s) * nbat

            @pl.when(s < ZT)
            def _zero():
                pltpu.sync_copy(z_ref.at[s], acc.at[pl.ds(s * ZW, ZW)])

            plsc.subcore_barrier()

            def bbody(b, carry):
                gb = tile * nbat + b
                pltpu.sync_copy(srcb_ref.at[gb], src_v)
                pltpu.sync_copy(dstb_ref.at[gb], dst_v)
                pltpu.sync_copy(exm_ref.at[exslot0 + b], ex_v)

                if hoff is not None:
                    def gbody(j, gc):
                        for k in range(CHK // 16):
                            src_v[j, pl.ds(16 * k, 16)] = (
                                src_v[j, pl.ds(16 * k, 16)] + hoff)
                        return gc
                    lax.fori_loop(0, BQ, gbody, 0)

                gcps = {}
                scps = {}
                gcps[0] = pltpu.async_copy(
                    x_ref.at[src_v.at[0]], rows_v.at[0], gsems[0])
                for j in range(BQ):
                    buf = j % 2
                    if j + 1 < BQ:
                        gcps[j + 1] = pltpu.async_copy(
                            x_ref.at[src_v.at[j + 1]], rows_v.at[1 - buf],
                            gsems[1 - buf])
                    gcps[j].wait()
                    if j >= 2:
                        scps[j - 2].wait()

                    def rbody(e, rc):
                        erow = jnp.full((16,), 0, jnp.int32) + (j * CHK + e)
                        bc = plsc.load_gather(ex_v, [erow])
                        for f in range(d // 16):
                            srow_v[buf, e, pl.ds(16 * f, 16)] = (
                                rows_v[buf, e, pl.ds(16 * f, 16)] * bc)
                        srow_v[buf, e, pl.ds(d, 16)] = bc
                        return rc
                    lax.fori_loop(0, CHK, rbody, 0, unroll=2)
                    scps[j] = pltpu.async_copy(
                        srow_v.at[buf], acc.at[dst_v.at[j]], ssems[buf],
                        add=True)
                scps[BQ - 2].wait()
                scps[BQ - 1].wait()
                return carry
            lax.fori_loop(0, nbat, bbody, 0)
            plsc.subcore_barrier()

            @pl.when(s < ZT)
            def _writeout():
                pltpu.sync_copy(acc.at[pl.ds(s * ZW, ZW)],
                                out_ref.at[pl.ds(out_base + s * ZW, ZW)])

            plsc.subcore_barrier()

    return scatter_kernel


_ex4 = _make_ex_sc(HEADS, edge_split=False)
_scat4 = _make_scatter_sc(HEADS, HID, edge_split=False)


def _edge4(x_flat, als_f, ald_f, srcm16, dstm16, srcb, dstb, zeros):
    exm = _ex4(als_f, ald_f, srcm16, dstm16)
    return _scat4(x_flat, exm.reshape(-1, BE), srcb, dstb, zeros)


_ex1 = _make_ex_sc(1, edge_split=True)
_scat1 = _make_scatter_sc(1, 16, edge_split=True)


def _edge1(x_flat, als_f, ald_f, srcm32, dstm32, srcb, dstb, zeros):
    exm = _ex1(als_f, ald_f, srcm32, dstm32)
    return _scat1(x_flat, exm.reshape(-1, BE), srcb, dstb, zeros)


_EF_NCHK = ECHUNKS // (NCORES * NSUB)


@functools.partial(
    pl.kernel,
    mesh=plsc.VectorSubcoreMesh(core_axis_name="c", subcore_axis_name="s"),
    out_type=jax.ShapeDtypeStruct((EB, 16), jnp.float32),
    compiler_params=pltpu.CompilerParams(use_tc_tiling_on_sc=False, needs_layout_passes=False),
    scratch_types=[
        pltpu.VMEM((_EF_NCHK, CHK), jnp.int32),
        pltpu.VMEM((_EF_NCHK, CHK), jnp.int32),
        pltpu.VMEM((2, CHK, 16), jnp.float32),
        pltpu.VMEM((2, CHK, 16), jnp.float32),
        pltpu.VMEM((2, CHK, 16), jnp.float32),
        pltpu.SemaphoreType.DMA,
        pltpu.SemaphoreType.DMA,
        pltpu.SemaphoreType.DMA,
        pltpu.SemaphoreType.DMA,
    ],
)
def _ef_kernel(p_ref, srcm_ref, dstm_ref, out_ref, src_v, dst_v,
               rs_v, rd_v, diff_v, gsem0, gsem1, wsem0, wsem1):
    c = lax.axis_index("c")
    s = lax.axis_index("s")
    w = c * NSUB + s
    cb = w * _EF_NCHK
    gsems = [gsem0, gsem1]
    wsems = [wsem0, wsem1]
    pltpu.sync_copy(srcm_ref.at[w], src_v)
    pltpu.sync_copy(dstm_ref.at[w], dst_v)

    def _issue(j, u):
        return (pltpu.async_copy(p_ref.at[src_v.at[j]], rs_v.at[u], gsems[u]),
                pltpu.async_copy(p_ref.at[dst_v.at[j]], rd_v.at[u], gsems[u]))

    def _finish(j, u, cps):
        cps[0].wait()
        cps[1].wait()

        def rbody(e, rc):
            diff_v[u, e] = rd_v[u, e] - rs_v[u, e]
            return rc
        lax.fori_loop(0, CHK, rbody, 0, unroll=2)
        return pltpu.async_copy(
            diff_v.at[u], out_ref.at[pl.ds((cb + j) * CHK, CHK)], wsems[u])

    def bbody(b, carry):
        g0 = _issue(b * 2, 0)
        g1 = _issue(b * 2 + 1, 1)
        w0 = _finish(b * 2, 0, g0)
        w1 = _finish(b * 2 + 1, 1, g1)
        w0.wait()
        w1.wait()
        return carry
    lax.fori_loop(0, _EF_NCHK // 2, bbody, 0)
    _finish(_EF_NCHK - 1, 0, _issue(_EF_NCHK - 1, 0)).wait()


def kernel(z, template_x, edge_index, mlp_w1, mlp_b1, mlp_w2, mlp_b2, mlp_w3, mlp_b3, gat1_w, gat1_as, gat1_ad, gat1_b, ln1_g, ln1_b, gat2_w, gat2_as, gat2_ad, gat2_b, ln2_g, ln2_b, gat3_w, gat3_as, gat3_ad, gat3_b):
    f = jnp.float32
    offs = jnp.arange(B, dtype=edge_index.dtype) * N
    src = (edge_index[0][None, :] + offs[:, None]).reshape(-1)
    dst = (edge_index[1][None, :] + offs[:, None]).reshape(-1)
    srcm16 = src.reshape(NSUB, EB // (NSUB * CHK), CHK)
    dstm16 = dst.reshape(NSUB, EB // (NSUB * CHK), CHK)
    srcm32 = src.reshape(NCORES * NSUB, _EF_NCHK, CHK)
    dstm32 = dst.reshape(NCORES * NSUB, _EF_NCHK, CHK)
    srcb = src.reshape(NBATCH, BQ, CHK)
    dstb = dst.reshape(NBATCH, BQ, CHK)
    zeros80 = jnp.zeros((ZT, ZW, HID + 16), f)
    zeros32 = jnp.zeros((ZT, ZW, 32), f)

    w1_heads = gat1_w.reshape(HEADS, HID, GIN)
    w2_heads = gat2_w.reshape(HEADS, HID, F1)
    w3pad = jnp.concatenate([gat3_w, jnp.zeros((3, F1), f)], axis=0).reshape(1, 16, F1)
    as3pad = jnp.concatenate([gat3_as, jnp.zeros((1, 3), f)], axis=1)
    ad3pad = jnp.concatenate([gat3_ad, jnp.zeros((1, 3), f)], axis=1)
    b3pad = jnp.concatenate([gat3_b, jnp.zeros((3,), f)]).reshape(1, 16)

    s = _style_mlp(z, mlp_w1, mlp_b1, mlp_w2, mlp_b2, mlp_w3, mlp_b3)

    x1_t, als1, ald1 = _tc0(template_x, s, w1_heads, gat1_as, gat1_ad)
    fused1 = _edge4(x1_t.reshape(HEADS * NB, HID),
                    als1.T.reshape(-1), ald1.T.reshape(-1),
                    srcm16, dstm16, srcb, dstb, zeros80)

    h1n, x2_t, als2, ald2 = _tcmid(
        fused1.reshape(HEADS, NB, HID + 16), gat1_b.reshape(1, F1),
        ln1_g.reshape(1, F1), ln1_b.reshape(1, F1),
        jnp.zeros((NB, F1), f), w2_heads, gat2_as, gat2_ad, residual=False)
    fused2 = _edge4(x2_t.reshape(HEADS * NB, HID),
                    als2.T.reshape(-1), ald2.T.reshape(-1),
                    srcm16, dstm16, srcb, dstb, zeros80)

    _, x3_t, als3, ald3 = _tcmid(
        fused2.reshape(HEADS, NB, HID + 16), gat2_b.reshape(1, F1),
        ln2_g.reshape(1, F1), ln2_b.reshape(1, F1),
        h1n, w3pad, as3pad.reshape(1, 16), ad3pad.reshape(1, 16),
        residual=True)
    fused3 = _edge1(x3_t.reshape(NB, 16),
                    als3.T.reshape(-1), ald3.T.reshape(-1),
                    srcm32, dstm32, srcb, dstb, zeros32)

    x_gen, p3, p16 = _tc3(fused3.reshape(2, NB, 32), b3pad, template_x)
    ef16 = _ef_kernel(p16, srcm32, dstm32)
    return x_gen, p3, ef16[:, :3]
